# Initial kernel scaffold; baseline (speedup 1.0000x reference)
import jax
import jax.numpy as jnp
from jax.experimental import pallas as pl


def kernel(loc_preds, cls_preds, priorbox, targets):
    raise NotImplementedError("write your pallas kernel here")


# speedup vs baseline: 38.8533x; 38.8533x over previous
import functools

import jax
import jax.numpy as jnp
from jax import lax
from jax.experimental import pallas as pl
from jax.experimental.pallas import tpu as pltpu
from jax.experimental.pallas import tpu_sc as plsc

B = 32
P = 32768
G = 16
NC = 2
NS = 16
CHUNK = 1024
NCHUNK = P // CHUNK
NVEC = CHUNK // 16
THR = 0.35
V0, V1 = 0.1, 0.2
NEGPOS = 3
NBIN = 4096
SUBSET = 8192
LN2 = 0.6931471805599453

f32 = jnp.float32
i32 = jnp.int32


def _i16():
    return lax.iota(i32, 16)


def _spl_f(x):
    return jnp.full((16,), x, f32)


def _spl_i(x):
    return jnp.full((16,), x, i32)


def _ln_m(x):
    s = (x - 1.0) / (x + 1.0)
    s2 = s * s
    return s * (2.0 + s2 * (2.0 / 3.0 + s2 * (2.0 / 5.0 + s2 * (2.0 / 7.0 + s2 * (2.0 / 9.0)))))


def _ln(x):
    b = lax.bitcast_convert_type(x, i32)
    e = (b >> 23) - 127
    m = lax.bitcast_convert_type((b & 0x7FFFFF) | 0x3F800000, f32)
    return e.astype(f32) * LN2 + _ln_m(m)


def _popcount(mask):
    return jnp.max(plsc.all_reduce_population_count(mask))


def _sc_body(pbt, loct, clst, tgt, pbf, locf, clsf, out,
             buf_pb, buf_loc, buf_cls, lc, tgv, gtc, hist_cnt, hist_sum,
             subset, gvec, idx_ref, outv, sem):
    wid = lax.axis_index("s") * NC + lax.axis_index("c")
    it16 = _i16()

    pltpu.sync_copy(tgt.at[wid], tgv)
    tw = [tgv[pl.ds(o * 16, 16)] for o in range(5)]

    def tsc(j):
        return tw[j // 16][j % 16]

    c0r = _spl_f(0.0)
    c1r = _spl_f(0.0)
    c2r = _spl_f(0.0)
    c3r = _spl_f(0.0)
    for g in range(G):
        t0 = tsc(5 * g)
        t1 = tsc(5 * g + 1)
        t2 = tsc(5 * g + 2)
        t3 = tsc(5 * g + 3)
        mg = it16 == g
        c0r = jnp.where(mg, t1, c0r)
        c1r = jnp.where(mg, t0, c1r)
        c2r = jnp.where(mg, t3, c2r)
        c3r = jnp.where(mg, t2, c3r)
        ag = (t3 - t1) * (t2 - t0)
        gtc[pl.ds((0 * G + g) * 16, 16)] = jnp.full((16,), t1, f32)
        gtc[pl.ds((1 * G + g) * 16, 16)] = jnp.full((16,), t0, f32)
        gtc[pl.ds((2 * G + g) * 16, 16)] = jnp.full((16,), t3, f32)
        gtc[pl.ds((3 * G + g) * 16, 16)] = jnp.full((16,), t2, f32)
        gtc[pl.ds((4 * G + g) * 16, 16)] = jnp.full((16,), ag, f32)
    def gt_bc(comp, g):
        return gtc[pl.ds((comp * G + g) * 16, 16)]

    def vec_body(ci, vi, st):
        bv, bi, a_sl1, a_ce, a_np = st
        off = vi * 16
        pidx = _spl_i(ci * CHUNK) + _spl_i(off) + it16
        cx = buf_pb[0, pl.ds(off, 16)]
        cy = buf_pb[1, pl.ds(off, 16)]
        w = buf_pb[2, pl.ds(off, 16)]
        h = buf_pb[3, pl.ds(off, 16)]
        px1 = cx - w / 2
        py1 = cy - h / 2
        px2 = cx + w / 2
        py2 = cy + h / 2
        parea = (px2 - px1) * (py2 - py1)
        btv = _spl_f(-1.0)
        m0 = _spl_f(0.0)
        m1 = _spl_f(0.0)
        m2 = _spl_f(0.0)
        m3 = _spl_f(0.0)
        bv2 = []
        bi2 = []
        for g in range(G):
            g0 = gt_bc(0, g)
            g1 = gt_bc(1, g)
            g2 = gt_bc(2, g)
            g3 = gt_bc(3, g)
            ga = gt_bc(4, g)
            wi = jnp.maximum(jnp.minimum(g2, px2) - jnp.maximum(g0, px1), 0.0)
            hi = jnp.maximum(jnp.minimum(g3, py2) - jnp.maximum(g1, py1), 0.0)
            inter = wi * hi
            iou = inter / (ga + parea - inter)
            m = iou > btv
            btv = jnp.where(m, iou, btv)
            m0 = jnp.where(m, g0, m0)
            m1 = jnp.where(m, g1, m1)
            m2 = jnp.where(m, g2, m2)
            m3 = jnp.where(m, g3, m3)
            mg = iou > bv[g]
            bv2.append(jnp.where(mg, iou, bv[g]))
            bi2.append(jnp.where(mg, pidx, bi[g]))
        pos = btv >= THR
        posf = jnp.where(pos, 1.0, 0.0)
        ecx = ((m0 + m2) / 2 - cx) / (V0 * w)
        ecy = ((m1 + m3) / 2 - cy) / (V0 * h)
        ew = _ln(jnp.maximum((m2 - m0) / w, 1e-8)) / V1
        eh = _ln(jnp.maximum((m3 - m1) / h, 1e-8)) / V1
        s = _spl_f(0.0)
        for comp, enc in ((0, ecx), (1, ecy), (2, ew), (3, eh)):
            d = buf_loc[comp, pl.ds(off, 16)] - enc
            ad = jnp.abs(d)
            s = s + jnp.where(ad < 1.0, 0.5 * d * d, ad - 0.5)
        a_sl1 = a_sl1 + s * posf
        x0 = buf_cls[0, pl.ds(off, 16)]
        x1 = buf_cls[1, pl.ds(off, 16)]
        mx = jnp.maximum(x0, x1)
        z = jnp.exp(-jnp.abs(x0 - x1))
        lse = mx + _ln_m(1.0 + z)
        ce1 = lse - x1
        ce0 = lse - x0
        a_ce = a_ce + jnp.where(pos, ce1, 0.0)
        a_np = a_np + jnp.where(pos, 1, 0)
        lc[pl.ds(ci * CHUNK + off, 16)] = jnp.where(pos, _spl_f(-0.0), ce0)
        return tuple(bv2), tuple(bi2), a_sl1, a_ce, a_np

    def chunk_body(ci, st):
        pltpu.sync_copy(pbt.at[:, pl.ds(ci * CHUNK, CHUNK)], buf_pb)
        pltpu.sync_copy(loct.at[wid, :, pl.ds(ci * CHUNK, CHUNK)], buf_loc)
        pltpu.sync_copy(clst.at[wid, :, pl.ds(ci * CHUNK, CHUNK)], buf_cls)
        return lax.fori_loop(0, NVEC, functools.partial(vec_body, ci), st, unroll=False)

    init = (tuple(_spl_f(-1.0) for _ in range(G)),
            tuple(_spl_i(0) for _ in range(G)),
            _spl_f(0.0), _spl_f(0.0), _spl_i(0))
    bv, bi, a_sl1, a_ce, a_np = lax.fori_loop(0, NCHUNK, chunk_body, init, unroll=False)

    bpi = _spl_i(0)
    pgs = []
    for g in range(G):
        gmax = jnp.max(bv[g])
        cand = jnp.where(bv[g] == gmax, bi[g], P)
        pg = jnp.min(cand)
        pgs.append(pg)
        bpi = jnp.where(it16 == g, pg, bpi)
    loser = it16 < 0
    for j in range(G):
        loser = loser | ((it16 < j) & (bpi == pgs[j]))
    winner = jnp.logical_not(loser)
    def fetch(hbm_flat, base):
        idx_ref[...] = bpi + _spl_i(base)
        pltpu.async_copy(hbm_flat.at[idx_ref], gvec, sem).wait()
        return gvec[...]

    pcx = fetch(pbf, 0)
    pcy = fetch(pbf, P)
    pw = fetch(pbf, 2 * P)
    ph = fetch(pbf, 3 * P)
    lp = [fetch(locf, (wid * 4 + c) * P) for c in range(4)]
    bx0 = fetch(clsf, wid * 2 * P)
    bx1 = fetch(clsf, (wid * 2 + 1) * P)
    px1 = pcx - pw / 2
    py1 = pcy - ph / 2
    px2 = pcx + pw / 2
    py2 = pcy + ph / 2
    parea = (px2 - px1) * (py2 - py1)
    obtv = _spl_f(-1.0)
    om0 = _spl_f(0.0)
    om1 = _spl_f(0.0)
    om2 = _spl_f(0.0)
    om3 = _spl_f(0.0)
    for g in range(G):
        g0 = gt_bc(0, g)
        g1 = gt_bc(1, g)
        g2 = gt_bc(2, g)
        g3 = gt_bc(3, g)
        ga = gt_bc(4, g)
        wi = jnp.maximum(jnp.minimum(g2, px2) - jnp.maximum(g0, px1), 0.0)
        hi = jnp.maximum(jnp.minimum(g3, py2) - jnp.maximum(g1, py1), 0.0)
        inter = wi * hi
        iou = inter / (ga + parea - inter)
        m = iou > obtv
        obtv = jnp.where(m, iou, obtv)
        om0 = jnp.where(m, g0, om0)
        om1 = jnp.where(m, g1, om1)
        om2 = jnp.where(m, g2, om2)
        om3 = jnp.where(m, g3, om3)
    old_pos = obtv >= THR

    def enc_sl1(m0, m1, m2, m3):
        ecx = ((m0 + m2) / 2 - pcx) / (V0 * pw)
        ecy = ((m1 + m3) / 2 - pcy) / (V0 * ph)
        ew = _ln(jnp.maximum((m2 - m0) / pw, 1e-8)) / V1
        eh = _ln(jnp.maximum((m3 - m1) / ph, 1e-8)) / V1
        s = _spl_f(0.0)
        for comp, enc in ((0, ecx), (1, ecy), (2, ew), (3, eh)):
            d = lp[comp] - enc
            ad = jnp.abs(d)
            s = s + jnp.where(ad < 1.0, 0.5 * d * d, ad - 0.5)
        return s

    sl1_new = enc_sl1(c0r, c1r, c2r, c3r)
    sl1_old = enc_sl1(om0, om1, om2, om3)
    z = jnp.exp(-jnp.abs(bx0 - bx1))
    lse = jnp.maximum(bx0, bx1) + _ln_m(1.0 + z)
    ce1 = lse - bx1
    newpos = winner & jnp.logical_not(old_pos)
    a_sl1 = a_sl1 + jnp.where(winner, sl1_new - jnp.where(old_pos, sl1_old, 0.0), 0.0)
    a_ce = a_ce + jnp.where(newpos, ce1, 0.0)
    a_np = a_np + jnp.where(newpos, 1, 0)
    plsc.store_scatter(lc, [bpi], _spl_f(-0.0), mask=winner)

    r_sl1 = jnp.sum(a_sl1)
    r_ce = jnp.sum(a_ce)
    r_np = jnp.sum(a_np)

    k = jnp.minimum(NEGPOS * r_np, P - 1)

    def zero_body(j, _):
        hist_cnt[pl.ds(j * 16, 16)] = _spl_i(0)
        hist_sum[pl.ds(j * 16, 16)] = _spl_f(0.0)
        return 0

    lax.fori_loop(0, NBIN // 16, zero_body, 0, unroll=False)

    def hist_body(j, _):
        v = lc[pl.ds(j * 16, 16)]
        mb = lax.bitcast_convert_type(v, i32) & 0x7FFFFFFF
        bn = mb >> 19
        plsc.addupdate_scatter(hist_cnt, [bn], _spl_i(1))
        plsc.addupdate_scatter(hist_sum, [bn], jnp.abs(v))
        return 0

    lax.fori_loop(0, P // 16, hist_body, 0, unroll=False)

    def scan_body(j, st):
        done, run_c, run_s, beta, cnt_hi, sum_hi = st
        jj = NBIN // 16 - 1 - j
        cv = hist_cnt[pl.ds(jj * 16, 16)]
        sv = hist_sum[pl.ds(jj * 16, 16)]
        s = jnp.sum(cv)
        found = jnp.logical_and(jnp.logical_not(done), run_c + s >= k)
        incl = plsc.cumsum(cv)
        cum_top = run_c + s - incl + cv
        maskl = cum_top >= k
        nb = _popcount(maskl)
        beta_lane = nb - 1
        beta_c = jj * 16 + beta_lane
        cnt_hi_c = run_c + jnp.sum(jnp.where(it16 > beta_lane, cv, 0))
        sum_hi_c = run_s + jnp.sum(jnp.where(it16 > beta_lane, sv, 0.0))
        beta = jnp.where(found, beta_c, beta)
        cnt_hi = jnp.where(found, cnt_hi_c, cnt_hi)
        sum_hi = jnp.where(found, sum_hi_c, sum_hi)
        done2 = jnp.logical_or(done, found)
        run_c = jnp.where(done2, run_c, run_c + s)
        run_s = jnp.where(done2, run_s, run_s + jnp.sum(sv))
        return done2, run_c, run_s, beta, cnt_hi, sum_hi

    _, _, _, beta, cnt_hi, sum_hi = lax.fori_loop(
        0, NBIN // 16, scan_body,
        (jnp.bool_(False), jnp.int32(0), jnp.float32(0.0),
         jnp.int32(0), jnp.int32(0), jnp.float32(0.0)), unroll=False)

    def subzero_body(j, _):
        subset[pl.ds(j * 16, 16)] = _spl_i(0)
        return 0

    lax.fori_loop(0, (SUBSET + 16) // 16, subzero_body, 0, unroll=False)

    def compact_body(j, off):
        v = lc[pl.ds(j * 16, 16)]
        mb = lax.bitcast_convert_type(v, i32) & 0x7FFFFFFF
        m = jnp.logical_and(mb >> 19 == beta, off < SUBSET)
        plsc.store_compressed(subset.at[pl.ds(off, 16)], mb, mask=m)
        return off + _popcount(m)

    s_cnt = lax.fori_loop(0, P // 16, compact_body, jnp.int32(0), unroll=False)
    nvec = (s_cnt + 15) // 16
    r = k - cnt_hi

    def count_ge(u):
        def cbody(i, acc):
            sv = subset[pl.ds(i * 16, 16)]
            valid = (it16 + i * 16) < s_cnt
            mm = jnp.logical_and(sv >= u, valid)
            return acc + _popcount(mm)
        return lax.fori_loop(0, nvec, cbody, jnp.int32(0))

    lo = beta << 19
    hi = lo | 0x7FFFF
    for _ in range(19):
        mid = lo + ((hi - lo + 1) >> 1)
        ge = count_ge(mid) >= r
        lo = jnp.where(ge, mid, lo)
        hi = jnp.where(ge, hi, mid - 1)
    tbits = lo
    tval = lax.bitcast_convert_type(tbits, f32)

    def above_body(i, st):
        c, sacc = st
        sv = subset[pl.ds(i * 16, 16)]
        valid = (it16 + i * 16) < s_cnt
        mm = jnp.logical_and(sv > tbits, valid)
        c = c + _popcount(mm)
        sacc = sacc + jnp.sum(jnp.where(mm, lax.bitcast_convert_type(sv, f32), 0.0))
        return c, sacc

    c2, s2 = lax.fori_loop(0, nvec, above_body, (jnp.int32(0), jnp.float32(0.0)))
    cnt_gt = cnt_hi + c2
    sum_gt = sum_hi + s2
    topk = sum_gt + (k - cnt_gt).astype(f32) * tval
    numer = r_ce + topk

    def edge_fn(_):
        need = k - cnt_gt

        def ebody(j, st):
            run, extra = st
            v = lc[pl.ds(j * 16, 16)]
            bb = lax.bitcast_convert_type(v, i32)
            zm = (bb & 0x7FFFFFFF) == 0
            incl = plsc.cumsum(jnp.where(zm, 1, 0))
            sel = jnp.logical_and(zm, (run + incl) <= need)
            extra = extra + _popcount(jnp.logical_and(sel, bb == 0))
            return run + _popcount(zm), extra

        _, extra = lax.fori_loop(0, P // 16, ebody, (jnp.int32(0), jnp.int32(0)))
        return r_np + cnt_gt + extra

    mask_cnt = lax.cond(tbits == 0, edge_fn, lambda _: r_np + k, 0)

    o = jnp.where(it16 == 0, r_sl1,
                  jnp.where(it16 == 1, r_np.astype(f32),
                            jnp.where(it16 == 2, numer,
                                      jnp.where(it16 == 3, mask_cnt.astype(f32), 0.0))))
    outv[...] = o
    pltpu.sync_copy(outv, out.at[wid])


@jax.jit
def kernel(loc_preds, cls_preds, priorbox, targets):
    pbt = priorbox.T
    loct = jnp.transpose(loc_preds, (0, 2, 1))
    clst = jnp.transpose(cls_preds, (0, 2, 1))
    pbf = pbt.reshape(4 * P)
    locf = loct.reshape(B * 4 * P)
    clsf = clst.reshape(B * 2 * P)
    tgf = targets.reshape(B, G * 5)
    mesh = plsc.VectorSubcoreMesh(core_axis_name="c", subcore_axis_name="s",
                                  num_cores=NC, num_subcores=NS)
    out = pl.kernel(
        _sc_body,
        out_type=jax.ShapeDtypeStruct((B, 16), f32),
        mesh=mesh,
        compiler_params=pltpu.CompilerParams(needs_layout_passes=False),
        scratch_types=[
            pltpu.VMEM((4, CHUNK), f32),
            pltpu.VMEM((4, CHUNK), f32),
            pltpu.VMEM((2, CHUNK), f32),
            pltpu.VMEM((P,), f32),
            pltpu.VMEM((G * 5,), f32),
            pltpu.VMEM((5 * G * 16,), f32),
            pltpu.VMEM((NBIN,), i32),
            pltpu.VMEM((NBIN,), f32),
            pltpu.VMEM((SUBSET + 16,), i32),
            pltpu.VMEM((16,), f32),
            pltpu.VMEM((16,), i32),
            pltpu.VMEM((16,), f32),
            pltpu.SemaphoreType.DMA,
        ],
    )(pbt, loct, clst, tgf, pbf, locf, clsf)
    tot = jnp.sum(out, axis=0)
    tot_sl1, tot_pos, tot_num, tot_mask = tot[0], tot[1], tot[2], tot[3]
    loss_loc = tot_sl1 / jnp.maximum(tot_pos * 4.0, 1.0)
    loss_cls = tot_num / jnp.maximum(tot_mask, 1.0)
    loss = (loss_cls + loss_loc) / jnp.maximum(tot_pos, 1.0)
    return (loss, loss_loc, loss_cls)


# speedup vs baseline: 53.1971x; 1.3692x over previous
import functools

import jax
import jax.numpy as jnp
from jax import lax
from jax.experimental import pallas as pl
from jax.experimental.pallas import tpu as pltpu
from jax.experimental.pallas import tpu_sc as plsc

B = 32
P = 32768
G = 16
NC = 2
NS = 16
CHUNK = 2048
NCHUNK = P // CHUNK
NVEC = CHUNK // 16
THR = 0.35
V0, V1 = 0.1, 0.2
NEGPOS = 3
NBIN = 4096
SUBSET = 8192
LN2 = 0.6931471805599453

f32 = jnp.float32
i32 = jnp.int32


def _i16():
    return lax.iota(i32, 16)


def _spl_f(x):
    return jnp.full((16,), x, f32)


def _spl_i(x):
    return jnp.full((16,), x, i32)


def _ln_m(x):
    s = (x - 1.0) / (x + 1.0)
    s2 = s * s
    return s * (2.0 + s2 * (2.0 / 3.0 + s2 * (2.0 / 5.0 + s2 * (2.0 / 7.0 + s2 * (2.0 / 9.0)))))


def _ln(x):
    b = lax.bitcast_convert_type(x, i32)
    e = (b >> 23) - 127
    m = lax.bitcast_convert_type((b & 0x7FFFFF) | 0x3F800000, f32)
    return e.astype(f32) * LN2 + _ln_m(m)


def _popcount(mask):
    return jnp.max(plsc.all_reduce_population_count(mask))


def _sc_body(pbt, loct, clst, tgt, pbf, locf, clsf, out,
             buf_pb, buf_loc, buf_cls, lc, tgv, gtc, hist_cnt, hist_sum,
             subset, gvec, idx_ref, outv, sem):
    wid = lax.axis_index("s") * NC + lax.axis_index("c")
    it16 = _i16()

    pltpu.sync_copy(tgt.at[wid], tgv)
    tw = [tgv[pl.ds(o * 16, 16)] for o in range(5)]

    def tsc(j):
        return tw[j // 16][j % 16]

    c0r = _spl_f(0.0)
    c1r = _spl_f(0.0)
    c2r = _spl_f(0.0)
    c3r = _spl_f(0.0)
    for g in range(G):
        t0 = tsc(5 * g)
        t1 = tsc(5 * g + 1)
        t2 = tsc(5 * g + 2)
        t3 = tsc(5 * g + 3)
        mg = it16 == g
        c0r = jnp.where(mg, t1, c0r)
        c1r = jnp.where(mg, t0, c1r)
        c2r = jnp.where(mg, t3, c2r)
        c3r = jnp.where(mg, t2, c3r)
        ag = (t3 - t1) * (t2 - t0)
        gtc[pl.ds((0 * G + g) * 16, 16)] = jnp.full((16,), t1, f32)
        gtc[pl.ds((1 * G + g) * 16, 16)] = jnp.full((16,), t0, f32)
        gtc[pl.ds((2 * G + g) * 16, 16)] = jnp.full((16,), t3, f32)
        gtc[pl.ds((3 * G + g) * 16, 16)] = jnp.full((16,), t2, f32)
        gtc[pl.ds((4 * G + g) * 16, 16)] = jnp.full((16,), ag, f32)
    def gt_bc(comp, g):
        return gtc[pl.ds((comp * G + g) * 16, 16)]

    def a1_body(ci, vi, st):
        a_sl1, a_ce, a_np = st
        off = vi * 16
        cx = buf_pb[0, pl.ds(off, 16)]
        cy = buf_pb[1, pl.ds(off, 16)]
        w = buf_pb[2, pl.ds(off, 16)]
        h = buf_pb[3, pl.ds(off, 16)]
        px1 = cx - w / 2
        py1 = cy - h / 2
        px2 = cx + w / 2
        py2 = cy + h / 2
        parea = (px2 - px1) * (py2 - py1)
        btv = _spl_f(-1.0)
        m0 = _spl_f(0.0)
        m1 = _spl_f(0.0)
        m2 = _spl_f(0.0)
        m3 = _spl_f(0.0)
        for g in range(G):
            g0 = gt_bc(0, g)
            g1 = gt_bc(1, g)
            g2 = gt_bc(2, g)
            g3 = gt_bc(3, g)
            ga = gt_bc(4, g)
            wi = jnp.maximum(jnp.minimum(g2, px2) - jnp.maximum(g0, px1), 0.0)
            hi = jnp.maximum(jnp.minimum(g3, py2) - jnp.maximum(g1, py1), 0.0)
            inter = wi * hi
            iou = inter / (ga + parea - inter)
            m = iou > btv
            btv = jnp.where(m, iou, btv)
            m0 = jnp.where(m, g0, m0)
            m1 = jnp.where(m, g1, m1)
            m2 = jnp.where(m, g2, m2)
            m3 = jnp.where(m, g3, m3)
        pos = btv >= THR
        posf = jnp.where(pos, 1.0, 0.0)
        ecx = ((m0 + m2) / 2 - cx) / (V0 * w)
        ecy = ((m1 + m3) / 2 - cy) / (V0 * h)
        ew = _ln(jnp.maximum((m2 - m0) / w, 1e-8)) / V1
        eh = _ln(jnp.maximum((m3 - m1) / h, 1e-8)) / V1
        s = _spl_f(0.0)
        for comp, enc in ((0, ecx), (1, ecy), (2, ew), (3, eh)):
            d = buf_loc[comp, pl.ds(off, 16)] - enc
            ad = jnp.abs(d)
            s = s + jnp.where(ad < 1.0, 0.5 * d * d, ad - 0.5)
        a_sl1 = a_sl1 + s * posf
        x0 = buf_cls[0, pl.ds(off, 16)]
        x1 = buf_cls[1, pl.ds(off, 16)]
        mx = jnp.maximum(x0, x1)
        z = jnp.exp(-jnp.abs(x0 - x1))
        lse = mx + _ln_m(1.0 + z)
        ce1 = lse - x1
        ce0 = lse - x0
        a_ce = a_ce + jnp.where(pos, ce1, 0.0)
        a_np = a_np + jnp.where(pos, 1, 0)
        lc[pl.ds(ci * CHUNK + off, 16)] = jnp.where(pos, _spl_f(-0.0), ce0)
        return a_sl1, a_ce, a_np

    def a2_body(ci, gbase, vi, st):
        bv8, bi8 = st
        off = vi * 16
        pidx = _spl_i(ci * CHUNK) + _spl_i(off) + it16
        cx = buf_pb[0, pl.ds(off, 16)]
        cy = buf_pb[1, pl.ds(off, 16)]
        w = buf_pb[2, pl.ds(off, 16)]
        h = buf_pb[3, pl.ds(off, 16)]
        px1 = cx - w / 2
        py1 = cy - h / 2
        px2 = cx + w / 2
        py2 = cy + h / 2
        parea = (px2 - px1) * (py2 - py1)
        bv2 = []
        bi2 = []
        for j in range(8):
            g = gbase + j
            g0 = gt_bc(0, g)
            g1 = gt_bc(1, g)
            g2 = gt_bc(2, g)
            g3 = gt_bc(3, g)
            ga = gt_bc(4, g)
            wi = jnp.maximum(jnp.minimum(g2, px2) - jnp.maximum(g0, px1), 0.0)
            hi = jnp.maximum(jnp.minimum(g3, py2) - jnp.maximum(g1, py1), 0.0)
            inter = wi * hi
            iou = inter / (ga + parea - inter)
            mg = iou > bv8[j]
            bv2.append(jnp.where(mg, iou, bv8[j]))
            bi2.append(jnp.where(mg, pidx, bi8[j]))
        return tuple(bv2), tuple(bi2)

    def chunk_body(ci, st):
        bv, bi, a_sl1, a_ce, a_np = st
        pltpu.sync_copy(pbt.at[:, pl.ds(ci * CHUNK, CHUNK)], buf_pb)
        pltpu.sync_copy(loct.at[wid, :, pl.ds(ci * CHUNK, CHUNK)], buf_loc)
        pltpu.sync_copy(clst.at[wid, :, pl.ds(ci * CHUNK, CHUNK)], buf_cls)
        a_sl1, a_ce, a_np = lax.fori_loop(
            0, NVEC, functools.partial(a1_body, ci),
            (a_sl1, a_ce, a_np), unroll=2)
        lo = lax.fori_loop(0, NVEC, functools.partial(a2_body, ci, 0),
                           (bv[:8], bi[:8]), unroll=2)
        hi = lax.fori_loop(0, NVEC, functools.partial(a2_body, ci, 8),
                           (bv[8:], bi[8:]), unroll=2)
        return (lo[0] + hi[0], lo[1] + hi[1], a_sl1, a_ce, a_np)

    init = (tuple(_spl_f(-1.0) for _ in range(G)),
            tuple(_spl_i(0) for _ in range(G)),
            _spl_f(0.0), _spl_f(0.0), _spl_i(0))
    bv, bi, a_sl1, a_ce, a_np = lax.fori_loop(0, NCHUNK, chunk_body, init, unroll=False)

    bpi = _spl_i(0)
    pgs = []
    for g in range(G):
        gmax = jnp.max(bv[g])
        cand = jnp.where(bv[g] == gmax, bi[g], P)
        pg = jnp.min(cand)
        pgs.append(pg)
        bpi = jnp.where(it16 == g, pg, bpi)
    loser = it16 < 0
    for j in range(G):
        loser = loser | ((it16 < j) & (bpi == pgs[j]))
    winner = jnp.logical_not(loser)
    def fetch(hbm_flat, base):
        idx_ref[...] = bpi + _spl_i(base)
        pltpu.async_copy(hbm_flat.at[idx_ref], gvec, sem).wait()
        return gvec[...]

    pcx = fetch(pbf, 0)
    pcy = fetch(pbf, P)
    pw = fetch(pbf, 2 * P)
    ph = fetch(pbf, 3 * P)
    lp = [fetch(locf, (wid * 4 + c) * P) for c in range(4)]
    bx0 = fetch(clsf, wid * 2 * P)
    bx1 = fetch(clsf, (wid * 2 + 1) * P)
    px1 = pcx - pw / 2
    py1 = pcy - ph / 2
    px2 = pcx + pw / 2
    py2 = pcy + ph / 2
    parea = (px2 - px1) * (py2 - py1)
    obtv = _spl_f(-1.0)
    om0 = _spl_f(0.0)
    om1 = _spl_f(0.0)
    om2 = _spl_f(0.0)
    om3 = _spl_f(0.0)
    for g in range(G):
        g0 = gt_bc(0, g)
        g1 = gt_bc(1, g)
        g2 = gt_bc(2, g)
        g3 = gt_bc(3, g)
        ga = gt_bc(4, g)
        wi = jnp.maximum(jnp.minimum(g2, px2) - jnp.maximum(g0, px1), 0.0)
        hi = jnp.maximum(jnp.minimum(g3, py2) - jnp.maximum(g1, py1), 0.0)
        inter = wi * hi
        iou = inter / (ga + parea - inter)
        m = iou > obtv
        obtv = jnp.where(m, iou, obtv)
        om0 = jnp.where(m, g0, om0)
        om1 = jnp.where(m, g1, om1)
        om2 = jnp.where(m, g2, om2)
        om3 = jnp.where(m, g3, om3)
    old_pos = obtv >= THR

    def enc_sl1(m0, m1, m2, m3):
        ecx = ((m0 + m2) / 2 - pcx) / (V0 * pw)
        ecy = ((m1 + m3) / 2 - pcy) / (V0 * ph)
        ew = _ln(jnp.maximum((m2 - m0) / pw, 1e-8)) / V1
        eh = _ln(jnp.maximum((m3 - m1) / ph, 1e-8)) / V1
        s = _spl_f(0.0)
        for comp, enc in ((0, ecx), (1, ecy), (2, ew), (3, eh)):
            d = lp[comp] - enc
            ad = jnp.abs(d)
            s = s + jnp.where(ad < 1.0, 0.5 * d * d, ad - 0.5)
        return s

    sl1_new = enc_sl1(c0r, c1r, c2r, c3r)
    sl1_old = enc_sl1(om0, om1, om2, om3)
    z = jnp.exp(-jnp.abs(bx0 - bx1))
    lse = jnp.maximum(bx0, bx1) + _ln_m(1.0 + z)
    ce1 = lse - bx1
    newpos = winner & jnp.logical_not(old_pos)
    a_sl1 = a_sl1 + jnp.where(winner, sl1_new - jnp.where(old_pos, sl1_old, 0.0), 0.0)
    a_ce = a_ce + jnp.where(newpos, ce1, 0.0)
    a_np = a_np + jnp.where(newpos, 1, 0)
    plsc.store_scatter(lc, [bpi], _spl_f(-0.0), mask=winner)

    r_sl1 = jnp.sum(a_sl1)
    r_ce = jnp.sum(a_ce)
    r_np = jnp.sum(a_np)

    k = jnp.minimum(NEGPOS * r_np, P - 1)

    def zero_body(j, _):
        hist_cnt[pl.ds(j * 16, 16)] = _spl_i(0)
        hist_sum[pl.ds(j * 16, 16)] = _spl_f(0.0)
        return 0

    lax.fori_loop(0, NBIN // 16, zero_body, 0, unroll=False)

    def hist_body(j, _):
        v = lc[pl.ds(j * 16, 16)]
        mb = lax.bitcast_convert_type(v, i32) & 0x7FFFFFFF
        bn = mb >> 19
        plsc.addupdate_scatter(hist_cnt, [bn], _spl_i(1))
        plsc.addupdate_scatter(hist_sum, [bn], jnp.abs(v))
        return 0

    lax.fori_loop(0, P // 16, hist_body, 0, unroll=False)

    def scan_body(j, st):
        done, run_c, run_s, beta, cnt_hi, sum_hi = st
        jj = NBIN // 16 - 1 - j
        cv = hist_cnt[pl.ds(jj * 16, 16)]
        sv = hist_sum[pl.ds(jj * 16, 16)]
        s = jnp.sum(cv)
        found = jnp.logical_and(jnp.logical_not(done), run_c + s >= k)
        incl = plsc.cumsum(cv)
        cum_top = run_c + s - incl + cv
        maskl = cum_top >= k
        nb = _popcount(maskl)
        beta_lane = nb - 1
        beta_c = jj * 16 + beta_lane
        cnt_hi_c = run_c + jnp.sum(jnp.where(it16 > beta_lane, cv, 0))
        sum_hi_c = run_s + jnp.sum(jnp.where(it16 > beta_lane, sv, 0.0))
        beta = jnp.where(found, beta_c, beta)
        cnt_hi = jnp.where(found, cnt_hi_c, cnt_hi)
        sum_hi = jnp.where(found, sum_hi_c, sum_hi)
        done2 = jnp.logical_or(done, found)
        run_c = jnp.where(done2, run_c, run_c + s)
        run_s = jnp.where(done2, run_s, run_s + jnp.sum(sv))
        return done2, run_c, run_s, beta, cnt_hi, sum_hi

    _, _, _, beta, cnt_hi, sum_hi = lax.fori_loop(
        0, NBIN // 16, scan_body,
        (jnp.bool_(False), jnp.int32(0), jnp.float32(0.0),
         jnp.int32(0), jnp.int32(0), jnp.float32(0.0)), unroll=False)

    def subzero_body(j, _):
        subset[pl.ds(j * 16, 16)] = _spl_i(0)
        return 0

    lax.fori_loop(0, (SUBSET + 16) // 16, subzero_body, 0, unroll=False)

    def compact_body(j, off):
        v = lc[pl.ds(j * 16, 16)]
        mb = lax.bitcast_convert_type(v, i32) & 0x7FFFFFFF
        m = jnp.logical_and(mb >> 19 == beta, off < SUBSET)
        plsc.store_compressed(subset.at[pl.ds(off, 16)], mb, mask=m)
        return off + _popcount(m)

    s_cnt = lax.fori_loop(0, P // 16, compact_body, jnp.int32(0), unroll=False)
    nvec = (s_cnt + 15) // 16
    r = k - cnt_hi

    def count_ge(u):
        def cbody(i, acc):
            sv = subset[pl.ds(i * 16, 16)]
            valid = (it16 + i * 16) < s_cnt
            mm = jnp.logical_and(sv >= u, valid)
            return acc + plsc.all_reduce_population_count(mm)
        return jnp.max(lax.fori_loop(0, nvec, cbody, _spl_i(0)))

    lo = beta << 19
    hi = lo | 0x7FFFF
    for _ in range(19):
        mid = lo + ((hi - lo + 1) >> 1)
        ge = count_ge(mid) >= r
        lo = jnp.where(ge, mid, lo)
        hi = jnp.where(ge, hi, mid - 1)
    tbits = lo
    tval = lax.bitcast_convert_type(tbits, f32)

    def above_body(i, st):
        c, sacc = st
        sv = subset[pl.ds(i * 16, 16)]
        valid = (it16 + i * 16) < s_cnt
        mm = jnp.logical_and(sv > tbits, valid)
        c = c + plsc.all_reduce_population_count(mm)
        sacc = sacc + jnp.where(mm, lax.bitcast_convert_type(sv, f32), 0.0)
        return c, sacc

    c2v, s2v = lax.fori_loop(0, nvec, above_body, (_spl_i(0), _spl_f(0.0)))
    c2 = jnp.max(c2v)
    s2 = jnp.sum(s2v)
    cnt_gt = cnt_hi + c2
    sum_gt = sum_hi + s2
    topk = sum_gt + (k - cnt_gt).astype(f32) * tval
    numer = r_ce + topk

    def edge_fn(_):
        need = k - cnt_gt

        def ebody(j, st):
            run, extra = st
            v = lc[pl.ds(j * 16, 16)]
            bb = lax.bitcast_convert_type(v, i32)
            zm = (bb & 0x7FFFFFFF) == 0
            incl = plsc.cumsum(jnp.where(zm, 1, 0))
            sel = jnp.logical_and(zm, (run + incl) <= need)
            extra = extra + _popcount(jnp.logical_and(sel, bb == 0))
            return run + _popcount(zm), extra

        _, extra = lax.fori_loop(0, P // 16, ebody, (jnp.int32(0), jnp.int32(0)))
        return r_np + cnt_gt + extra

    mask_cnt = lax.cond(tbits == 0, edge_fn, lambda _: r_np + k, 0)

    o = jnp.where(it16 == 0, r_sl1,
                  jnp.where(it16 == 1, r_np.astype(f32),
                            jnp.where(it16 == 2, numer,
                                      jnp.where(it16 == 3, mask_cnt.astype(f32), 0.0))))
    outv[...] = o
    pltpu.sync_copy(outv, out.at[wid])


@jax.jit
def kernel(loc_preds, cls_preds, priorbox, targets):
    pbt = priorbox.T
    loct = jnp.transpose(loc_preds, (0, 2, 1))
    clst = jnp.transpose(cls_preds, (0, 2, 1))
    pbf = pbt.reshape(4 * P)
    locf = loct.reshape(B * 4 * P)
    clsf = clst.reshape(B * 2 * P)
    tgf = targets.reshape(B, G * 5)
    mesh = plsc.VectorSubcoreMesh(core_axis_name="c", subcore_axis_name="s",
                                  num_cores=NC, num_subcores=NS)
    out = pl.kernel(
        _sc_body,
        out_type=jax.ShapeDtypeStruct((B, 16), f32),
        mesh=mesh,
        compiler_params=pltpu.CompilerParams(needs_layout_passes=False),
        scratch_types=[
            pltpu.VMEM((4, CHUNK), f32),
            pltpu.VMEM((4, CHUNK), f32),
            pltpu.VMEM((2, CHUNK), f32),
            pltpu.VMEM((P,), f32),
            pltpu.VMEM((G * 5,), f32),
            pltpu.VMEM((5 * G * 16,), f32),
            pltpu.VMEM((NBIN,), i32),
            pltpu.VMEM((NBIN,), f32),
            pltpu.VMEM((SUBSET + 16,), i32),
            pltpu.VMEM((16,), f32),
            pltpu.VMEM((16,), i32),
            pltpu.VMEM((16,), f32),
            pltpu.SemaphoreType.DMA,
        ],
    )(pbt, loct, clst, tgf, pbf, locf, clsf)
    tot = jnp.sum(out, axis=0)
    tot_sl1, tot_pos, tot_num, tot_mask = tot[0], tot[1], tot[2], tot[3]
    loss_loc = tot_sl1 / jnp.maximum(tot_pos * 4.0, 1.0)
    loss_cls = tot_num / jnp.maximum(tot_mask, 1.0)
    loss = (loss_cls + loss_loc) / jnp.maximum(tot_pos, 1.0)
    return (loss, loss_loc, loss_cls)


# speedup vs baseline: 56.2265x; 1.0569x over previous
import functools

import jax
import jax.numpy as jnp
from jax import lax
from jax.experimental import pallas as pl
from jax.experimental.pallas import tpu as pltpu
from jax.experimental.pallas import tpu_sc as plsc

B = 32
P = 32768
G = 16
NC = 2
NS = 16
CHUNK = 2048
NCHUNK = P // CHUNK
NVEC = CHUNK // 16
THR = 0.35
V0, V1 = 0.1, 0.2
NEGPOS = 3
NBIN = 4096
SUBSET = 8192
LN2 = 0.6931471805599453

f32 = jnp.float32
i32 = jnp.int32


def _i16():
    return lax.iota(i32, 16)


def _spl_f(x):
    return jnp.full((16,), x, f32)


def _spl_i(x):
    return jnp.full((16,), x, i32)


def _ln_m(x):
    s = (x - 1.0) / (x + 1.0)
    s2 = s * s
    return s * (2.0 + s2 * (2.0 / 3.0 + s2 * (2.0 / 5.0 + s2 * (2.0 / 7.0 + s2 * (2.0 / 9.0)))))


def _ln(x):
    b = lax.bitcast_convert_type(x, i32)
    e = (b >> 23) - 127
    m = lax.bitcast_convert_type((b & 0x7FFFFF) | 0x3F800000, f32)
    return e.astype(f32) * LN2 + _ln_m(m)


def _popcount(mask):
    return jnp.max(plsc.all_reduce_population_count(mask))


def _sc_body(pbt, loct, clst, tgt, pbf, locf, clsf, out,
             buf_pb, buf_loc, buf_cls, lc, tgv, gtc, hist_cnt, hist_sum,
             subset, gvec, idx_ref, outv, mbuf, sem):
    wid = lax.axis_index("s") * NC + lax.axis_index("c")
    it16 = _i16()

    pltpu.sync_copy(tgt.at[wid], tgv)
    tw = [tgv[pl.ds(o * 16, 16)] for o in range(5)]

    def tsc(j):
        return tw[j // 16][j % 16]

    c0r = _spl_f(0.0)
    c1r = _spl_f(0.0)
    c2r = _spl_f(0.0)
    c3r = _spl_f(0.0)
    for g in range(G):
        t0 = tsc(5 * g)
        t1 = tsc(5 * g + 1)
        t2 = tsc(5 * g + 2)
        t3 = tsc(5 * g + 3)
        mg = it16 == g
        c0r = jnp.where(mg, t1, c0r)
        c1r = jnp.where(mg, t0, c1r)
        c2r = jnp.where(mg, t3, c2r)
        c3r = jnp.where(mg, t2, c3r)
        ag = (t3 - t1) * (t2 - t0)
        gtc[pl.ds((0 * G + g) * 16, 16)] = jnp.full((16,), t1, f32)
        gtc[pl.ds((1 * G + g) * 16, 16)] = jnp.full((16,), t0, f32)
        gtc[pl.ds((2 * G + g) * 16, 16)] = jnp.full((16,), t3, f32)
        gtc[pl.ds((3 * G + g) * 16, 16)] = jnp.full((16,), t2, f32)
        gtc[pl.ds((4 * G + g) * 16, 16)] = jnp.full((16,), ag, f32)
    def gt_bc(comp, g):
        return gtc[pl.ds((comp * G + g) * 16, 16)]

    def a1_body(ci, vi, st):
        off = vi * 16
        cx = buf_pb[0, pl.ds(off, 16)]
        cy = buf_pb[1, pl.ds(off, 16)]
        w = buf_pb[2, pl.ds(off, 16)]
        h = buf_pb[3, pl.ds(off, 16)]
        px1 = cx - w / 2
        py1 = cy - h / 2
        px2 = cx + w / 2
        py2 = cy + h / 2
        parea = (px2 - px1) * (py2 - py1)
        btv = _spl_f(-1.0)
        m0 = _spl_f(0.0)
        m1 = _spl_f(0.0)
        m2 = _spl_f(0.0)
        m3 = _spl_f(0.0)
        for g in range(G):
            g0 = gt_bc(0, g)
            g1 = gt_bc(1, g)
            g2 = gt_bc(2, g)
            g3 = gt_bc(3, g)
            ga = gt_bc(4, g)
            wi = jnp.maximum(jnp.minimum(g2, px2) - jnp.maximum(g0, px1), 0.0)
            hi = jnp.maximum(jnp.minimum(g3, py2) - jnp.maximum(g1, py1), 0.0)
            inter = wi * hi
            iou = inter / (ga + parea - inter)
            m = iou > btv
            btv = jnp.where(m, iou, btv)
            m0 = jnp.where(m, g0, m0)
            m1 = jnp.where(m, g1, m1)
            m2 = jnp.where(m, g2, m2)
            m3 = jnp.where(m, g3, m3)
        mbuf[0, pl.ds(off, 16)] = btv
        mbuf[1, pl.ds(off, 16)] = m0
        mbuf[2, pl.ds(off, 16)] = m1
        mbuf[3, pl.ds(off, 16)] = m2
        mbuf[4, pl.ds(off, 16)] = m3
        return 0

    def a1c_body(ci, vi, st):
        a_sl1, a_ce, a_np = st
        off = vi * 16
        cx = buf_pb[0, pl.ds(off, 16)]
        cy = buf_pb[1, pl.ds(off, 16)]
        w = buf_pb[2, pl.ds(off, 16)]
        h = buf_pb[3, pl.ds(off, 16)]
        btv = mbuf[0, pl.ds(off, 16)]
        m0 = mbuf[1, pl.ds(off, 16)]
        m1 = mbuf[2, pl.ds(off, 16)]
        m2 = mbuf[3, pl.ds(off, 16)]
        m3 = mbuf[4, pl.ds(off, 16)]
        pos = btv >= THR
        posf = jnp.where(pos, 1.0, 0.0)
        ecx = ((m0 + m2) / 2 - cx) / (V0 * w)
        ecy = ((m1 + m3) / 2 - cy) / (V0 * h)
        ew = _ln(jnp.maximum((m2 - m0) / w, 1e-8)) / V1
        eh = _ln(jnp.maximum((m3 - m1) / h, 1e-8)) / V1
        s = _spl_f(0.0)
        for comp, enc in ((0, ecx), (1, ecy), (2, ew), (3, eh)):
            d = buf_loc[comp, pl.ds(off, 16)] - enc
            ad = jnp.abs(d)
            s = s + jnp.where(ad < 1.0, 0.5 * d * d, ad - 0.5)
        a_sl1 = a_sl1 + s * posf
        x0 = buf_cls[0, pl.ds(off, 16)]
        x1 = buf_cls[1, pl.ds(off, 16)]
        mx = jnp.maximum(x0, x1)
        z = jnp.exp(-jnp.abs(x0 - x1))
        lse = mx + _ln_m(1.0 + z)
        ce1 = lse - x1
        ce0 = lse - x0
        a_ce = a_ce + jnp.where(pos, ce1, 0.0)
        a_np = a_np + jnp.where(pos, 1, 0)
        lc[pl.ds(ci * CHUNK + off, 16)] = jnp.where(pos, _spl_f(-0.0), ce0)
        return a_sl1, a_ce, a_np

    def a2_body(ci, gbase, vi, st):
        bv8, bi8 = st
        off = vi * 16
        pidx = _spl_i(ci * CHUNK) + _spl_i(off) + it16
        cx = buf_pb[0, pl.ds(off, 16)]
        cy = buf_pb[1, pl.ds(off, 16)]
        w = buf_pb[2, pl.ds(off, 16)]
        h = buf_pb[3, pl.ds(off, 16)]
        px1 = cx - w / 2
        py1 = cy - h / 2
        px2 = cx + w / 2
        py2 = cy + h / 2
        parea = (px2 - px1) * (py2 - py1)
        bv2 = []
        bi2 = []
        for j in range(8):
            g = gbase + j
            g0 = gt_bc(0, g)
            g1 = gt_bc(1, g)
            g2 = gt_bc(2, g)
            g3 = gt_bc(3, g)
            ga = gt_bc(4, g)
            wi = jnp.maximum(jnp.minimum(g2, px2) - jnp.maximum(g0, px1), 0.0)
            hi = jnp.maximum(jnp.minimum(g3, py2) - jnp.maximum(g1, py1), 0.0)
            inter = wi * hi
            iou = inter / (ga + parea - inter)
            mg = iou > bv8[j]
            bv2.append(jnp.where(mg, iou, bv8[j]))
            bi2.append(jnp.where(mg, pidx, bi8[j]))
        return tuple(bv2), tuple(bi2)

    def chunk_body(ci, st):
        bv, bi, a_sl1, a_ce, a_np = st
        pltpu.sync_copy(pbt.at[:, pl.ds(ci * CHUNK, CHUNK)], buf_pb)
        pltpu.sync_copy(loct.at[wid, :, pl.ds(ci * CHUNK, CHUNK)], buf_loc)
        pltpu.sync_copy(clst.at[wid, :, pl.ds(ci * CHUNK, CHUNK)], buf_cls)
        lax.fori_loop(0, NVEC, functools.partial(a1_body, ci), 0, unroll=2)
        a_sl1, a_ce, a_np = lax.fori_loop(
            0, NVEC, functools.partial(a1c_body, ci),
            (a_sl1, a_ce, a_np), unroll=2)
        lo = lax.fori_loop(0, NVEC, functools.partial(a2_body, ci, 0),
                           (bv[:8], bi[:8]), unroll=2)
        hi = lax.fori_loop(0, NVEC, functools.partial(a2_body, ci, 8),
                           (bv[8:], bi[8:]), unroll=2)
        return (lo[0] + hi[0], lo[1] + hi[1], a_sl1, a_ce, a_np)

    init = (tuple(_spl_f(-1.0) for _ in range(G)),
            tuple(_spl_i(0) for _ in range(G)),
            _spl_f(0.0), _spl_f(0.0), _spl_i(0))
    bv, bi, a_sl1, a_ce, a_np = lax.fori_loop(0, NCHUNK, chunk_body, init, unroll=False)

    bpi = _spl_i(0)
    pgs = []
    for g in range(G):
        gmax = jnp.max(bv[g])
        cand = jnp.where(bv[g] == gmax, bi[g], P)
        pg = jnp.min(cand)
        pgs.append(pg)
        bpi = jnp.where(it16 == g, pg, bpi)
    loser = it16 < 0
    for j in range(G):
        loser = loser | ((it16 < j) & (bpi == pgs[j]))
    winner = jnp.logical_not(loser)
    def fetch(hbm_flat, base):
        idx_ref[...] = bpi + _spl_i(base)
        pltpu.async_copy(hbm_flat.at[idx_ref], gvec, sem).wait()
        return gvec[...]

    pcx = fetch(pbf, 0)
    pcy = fetch(pbf, P)
    pw = fetch(pbf, 2 * P)
    ph = fetch(pbf, 3 * P)
    lp = [fetch(locf, (wid * 4 + c) * P) for c in range(4)]
    bx0 = fetch(clsf, wid * 2 * P)
    bx1 = fetch(clsf, (wid * 2 + 1) * P)
    px1 = pcx - pw / 2
    py1 = pcy - ph / 2
    px2 = pcx + pw / 2
    py2 = pcy + ph / 2
    parea = (px2 - px1) * (py2 - py1)
    obtv = _spl_f(-1.0)
    om0 = _spl_f(0.0)
    om1 = _spl_f(0.0)
    om2 = _spl_f(0.0)
    om3 = _spl_f(0.0)
    for g in range(G):
        g0 = gt_bc(0, g)
        g1 = gt_bc(1, g)
        g2 = gt_bc(2, g)
        g3 = gt_bc(3, g)
        ga = gt_bc(4, g)
        wi = jnp.maximum(jnp.minimum(g2, px2) - jnp.maximum(g0, px1), 0.0)
        hi = jnp.maximum(jnp.minimum(g3, py2) - jnp.maximum(g1, py1), 0.0)
        inter = wi * hi
        iou = inter / (ga + parea - inter)
        m = iou > obtv
        obtv = jnp.where(m, iou, obtv)
        om0 = jnp.where(m, g0, om0)
        om1 = jnp.where(m, g1, om1)
        om2 = jnp.where(m, g2, om2)
        om3 = jnp.where(m, g3, om3)
    old_pos = obtv >= THR

    def enc_sl1(m0, m1, m2, m3):
        ecx = ((m0 + m2) / 2 - pcx) / (V0 * pw)
        ecy = ((m1 + m3) / 2 - pcy) / (V0 * ph)
        ew = _ln(jnp.maximum((m2 - m0) / pw, 1e-8)) / V1
        eh = _ln(jnp.maximum((m3 - m1) / ph, 1e-8)) / V1
        s = _spl_f(0.0)
        for comp, enc in ((0, ecx), (1, ecy), (2, ew), (3, eh)):
            d = lp[comp] - enc
            ad = jnp.abs(d)
            s = s + jnp.where(ad < 1.0, 0.5 * d * d, ad - 0.5)
        return s

    sl1_new = enc_sl1(c0r, c1r, c2r, c3r)
    sl1_old = enc_sl1(om0, om1, om2, om3)
    z = jnp.exp(-jnp.abs(bx0 - bx1))
    lse = jnp.maximum(bx0, bx1) + _ln_m(1.0 + z)
    ce1 = lse - bx1
    newpos = winner & jnp.logical_not(old_pos)
    a_sl1 = a_sl1 + jnp.where(winner, sl1_new - jnp.where(old_pos, sl1_old, 0.0), 0.0)
    a_ce = a_ce + jnp.where(newpos, ce1, 0.0)
    a_np = a_np + jnp.where(newpos, 1, 0)
    plsc.store_scatter(lc, [bpi], _spl_f(-0.0), mask=winner)

    r_sl1 = jnp.sum(a_sl1)
    r_ce = jnp.sum(a_ce)
    r_np = jnp.sum(a_np)

    k = jnp.minimum(NEGPOS * r_np, P - 1)

    def zero_body(j, _):
        hist_cnt[pl.ds(j * 16, 16)] = _spl_i(0)
        hist_sum[pl.ds(j * 16, 16)] = _spl_f(0.0)
        return 0

    lax.fori_loop(0, NBIN // 16, zero_body, 0, unroll=False)

    def hist_body(j, _):
        v = lc[pl.ds(j * 16, 16)]
        mb = lax.bitcast_convert_type(v, i32) & 0x7FFFFFFF
        bn = mb >> 19
        plsc.addupdate_scatter(hist_cnt, [bn], _spl_i(1))
        plsc.addupdate_scatter(hist_sum, [bn], jnp.abs(v))
        return 0

    lax.fori_loop(0, P // 16, hist_body, 0, unroll=False)

    def scan_body(j, st):
        done, run_c, run_s, beta, cnt_hi, sum_hi = st
        jj = NBIN // 16 - 1 - j
        cv = hist_cnt[pl.ds(jj * 16, 16)]
        sv = hist_sum[pl.ds(jj * 16, 16)]
        s = jnp.sum(cv)
        found = jnp.logical_and(jnp.logical_not(done), run_c + s >= k)
        incl = plsc.cumsum(cv)
        cum_top = run_c + s - incl + cv
        maskl = cum_top >= k
        nb = _popcount(maskl)
        beta_lane = nb - 1
        beta_c = jj * 16 + beta_lane
        cnt_hi_c = run_c + jnp.sum(jnp.where(it16 > beta_lane, cv, 0))
        sum_hi_c = run_s + jnp.sum(jnp.where(it16 > beta_lane, sv, 0.0))
        beta = jnp.where(found, beta_c, beta)
        cnt_hi = jnp.where(found, cnt_hi_c, cnt_hi)
        sum_hi = jnp.where(found, sum_hi_c, sum_hi)
        done2 = jnp.logical_or(done, found)
        run_c = jnp.where(done2, run_c, run_c + s)
        run_s = jnp.where(done2, run_s, run_s + jnp.sum(sv))
        return done2, run_c, run_s, beta, cnt_hi, sum_hi

    _, _, _, beta, cnt_hi, sum_hi = lax.fori_loop(
        0, NBIN // 16, scan_body,
        (jnp.bool_(False), jnp.int32(0), jnp.float32(0.0),
         jnp.int32(0), jnp.int32(0), jnp.float32(0.0)), unroll=False)

    def subzero_body(j, _):
        subset[pl.ds(j * 16, 16)] = _spl_i(0)
        return 0

    lax.fori_loop(0, (SUBSET + 16) // 16, subzero_body, 0, unroll=False)

    def compact_body(j, off):
        v = lc[pl.ds(j * 16, 16)]
        mb = lax.bitcast_convert_type(v, i32) & 0x7FFFFFFF
        m = jnp.logical_and(mb >> 19 == beta, off < SUBSET)
        plsc.store_compressed(subset.at[pl.ds(off, 16)], mb, mask=m)
        return off + _popcount(m)

    s_cnt = lax.fori_loop(0, P // 16, compact_body, jnp.int32(0), unroll=False)
    nvec = (s_cnt + 15) // 16
    r = k - cnt_hi

    def count_ge(u):
        def cbody(i, acc):
            sv = subset[pl.ds(i * 16, 16)]
            valid = (it16 + i * 16) < s_cnt
            mm = jnp.logical_and(sv >= u, valid)
            return acc + plsc.all_reduce_population_count(mm)
        return jnp.max(lax.fori_loop(0, nvec, cbody, _spl_i(0)))

    lo = beta << 19
    hi = lo | 0x7FFFF
    for _ in range(19):
        mid = lo + ((hi - lo + 1) >> 1)
        ge = count_ge(mid) >= r
        lo = jnp.where(ge, mid, lo)
        hi = jnp.where(ge, hi, mid - 1)
    tbits = lo
    tval = lax.bitcast_convert_type(tbits, f32)

    def above_body(i, st):
        c, sacc = st
        sv = subset[pl.ds(i * 16, 16)]
        valid = (it16 + i * 16) < s_cnt
        mm = jnp.logical_and(sv > tbits, valid)
        c = c + plsc.all_reduce_population_count(mm)
        sacc = sacc + jnp.where(mm, lax.bitcast_convert_type(sv, f32), 0.0)
        return c, sacc

    c2v, s2v = lax.fori_loop(0, nvec, above_body, (_spl_i(0), _spl_f(0.0)))
    c2 = jnp.max(c2v)
    s2 = jnp.sum(s2v)
    cnt_gt = cnt_hi + c2
    sum_gt = sum_hi + s2
    topk = sum_gt + (k - cnt_gt).astype(f32) * tval
    numer = r_ce + topk

    def edge_fn(_):
        need = k - cnt_gt

        def ebody(j, st):
            run, extra = st
            v = lc[pl.ds(j * 16, 16)]
            bb = lax.bitcast_convert_type(v, i32)
            zm = (bb & 0x7FFFFFFF) == 0
            incl = plsc.cumsum(jnp.where(zm, 1, 0))
            sel = jnp.logical_and(zm, (run + incl) <= need)
            extra = extra + _popcount(jnp.logical_and(sel, bb == 0))
            return run + _popcount(zm), extra

        _, extra = lax.fori_loop(0, P // 16, ebody, (jnp.int32(0), jnp.int32(0)))
        return r_np + cnt_gt + extra

    mask_cnt = lax.cond(tbits == 0, edge_fn, lambda _: r_np + k, 0)

    o = jnp.where(it16 == 0, r_sl1,
                  jnp.where(it16 == 1, r_np.astype(f32),
                            jnp.where(it16 == 2, numer,
                                      jnp.where(it16 == 3, mask_cnt.astype(f32), 0.0))))
    outv[...] = o
    pltpu.sync_copy(outv, out.at[wid])


@jax.jit
def kernel(loc_preds, cls_preds, priorbox, targets):
    pbt = priorbox.T
    loct = jnp.transpose(loc_preds, (0, 2, 1))
    clst = jnp.transpose(cls_preds, (0, 2, 1))
    pbf = pbt.reshape(4 * P)
    locf = loct.reshape(B * 4 * P)
    clsf = clst.reshape(B * 2 * P)
    tgf = targets.reshape(B, G * 5)
    mesh = plsc.VectorSubcoreMesh(core_axis_name="c", subcore_axis_name="s",
                                  num_cores=NC, num_subcores=NS)
    out = pl.kernel(
        _sc_body,
        out_type=jax.ShapeDtypeStruct((B, 16), f32),
        mesh=mesh,
        compiler_params=pltpu.CompilerParams(needs_layout_passes=False),
        scratch_types=[
            pltpu.VMEM((4, CHUNK), f32),
            pltpu.VMEM((4, CHUNK), f32),
            pltpu.VMEM((2, CHUNK), f32),
            pltpu.VMEM((P,), f32),
            pltpu.VMEM((G * 5,), f32),
            pltpu.VMEM((5 * G * 16,), f32),
            pltpu.VMEM((NBIN,), i32),
            pltpu.VMEM((NBIN,), f32),
            pltpu.VMEM((SUBSET + 16,), i32),
            pltpu.VMEM((16,), f32),
            pltpu.VMEM((16,), i32),
            pltpu.VMEM((16,), f32),
            pltpu.VMEM((5, CHUNK), f32),
            pltpu.SemaphoreType.DMA,
        ],
    )(pbt, loct, clst, tgf, pbf, locf, clsf)
    tot = jnp.sum(out, axis=0)
    tot_sl1, tot_pos, tot_num, tot_mask = tot[0], tot[1], tot[2], tot[3]
    loss_loc = tot_sl1 / jnp.maximum(tot_pos * 4.0, 1.0)
    loss_cls = tot_num / jnp.maximum(tot_mask, 1.0)
    loss = (loss_cls + loss_loc) / jnp.maximum(tot_pos, 1.0)
    return (loss, loss_loc, loss_cls)


# speedup vs baseline: 61.8329x; 1.0997x over previous
import functools

import jax
import jax.numpy as jnp
from jax import lax
from jax.experimental import pallas as pl
from jax.experimental.pallas import tpu as pltpu
from jax.experimental.pallas import tpu_sc as plsc

B = 32
P = 32768
G = 16
NC = 2
NS = 16
CHUNK = 2048
NCHUNK = P // CHUNK
NVEC = CHUNK // 16
THR = 0.35
V0, V1 = 0.1, 0.2
NEGPOS = 3
NBIN = 4096
SUBSET = 8192
LN2 = 0.6931471805599453

f32 = jnp.float32
i32 = jnp.int32


def _i16():
    return lax.iota(i32, 16)


def _spl_f(x):
    return jnp.full((16,), x, f32)


def _spl_i(x):
    return jnp.full((16,), x, i32)


def _ln_m(x):
    s = (x - 1.0) / (x + 1.0)
    s2 = s * s
    return s * (2.0 + s2 * (2.0 / 3.0 + s2 * (2.0 / 5.0 + s2 * (2.0 / 7.0 + s2 * (2.0 / 9.0)))))


def _ln(x):
    b = lax.bitcast_convert_type(x, i32)
    e = (b >> 23) - 127
    m = lax.bitcast_convert_type((b & 0x7FFFFF) | 0x3F800000, f32)
    return e.astype(f32) * LN2 + _ln_m(m)


def _popcount(mask):
    return jnp.max(plsc.all_reduce_population_count(mask))


def _sc_body(pbt, loct, clst, tgt, pbf, locf, clsf, out,
             buf_pb, buf_loc, buf_cls, lc, tgv, gtc, hist_cnt, hist_sum,
             subset, gvec, idx_ref, outv, mbuf, sem):
    wid = lax.axis_index("s") * NC + lax.axis_index("c")
    it16 = _i16()

    pltpu.sync_copy(tgt.at[wid], tgv)
    tw = [tgv[pl.ds(o * 16, 16)] for o in range(5)]

    def tsc(j):
        return tw[j // 16][j % 16]

    c0r = _spl_f(0.0)
    c1r = _spl_f(0.0)
    c2r = _spl_f(0.0)
    c3r = _spl_f(0.0)
    for g in range(G):
        t0 = tsc(5 * g)
        t1 = tsc(5 * g + 1)
        t2 = tsc(5 * g + 2)
        t3 = tsc(5 * g + 3)
        mg = it16 == g
        c0r = jnp.where(mg, t1, c0r)
        c1r = jnp.where(mg, t0, c1r)
        c2r = jnp.where(mg, t3, c2r)
        c3r = jnp.where(mg, t2, c3r)
        ag = (t3 - t1) * (t2 - t0)
        gtc[pl.ds((0 * G + g) * 16, 16)] = jnp.full((16,), t1, f32)
        gtc[pl.ds((1 * G + g) * 16, 16)] = jnp.full((16,), t0, f32)
        gtc[pl.ds((2 * G + g) * 16, 16)] = jnp.full((16,), t3, f32)
        gtc[pl.ds((3 * G + g) * 16, 16)] = jnp.full((16,), t2, f32)
        gtc[pl.ds((4 * G + g) * 16, 16)] = jnp.full((16,), ag, f32)
    def gt_bc(comp, g):
        return gtc[pl.ds((comp * G + g) * 16, 16)]

    def a1_body(ci, vi, st):
        bv8, bi8 = st
        off = vi * 16
        pidx = _spl_i(ci * CHUNK) + _spl_i(off) + it16
        cx = buf_pb[0, pl.ds(off, 16)]
        cy = buf_pb[1, pl.ds(off, 16)]
        w = buf_pb[2, pl.ds(off, 16)]
        h = buf_pb[3, pl.ds(off, 16)]
        px1 = cx - w / 2
        py1 = cy - h / 2
        px2 = cx + w / 2
        py2 = cy + h / 2
        parea = (px2 - px1) * (py2 - py1)
        btv = _spl_f(-1.0)
        m0 = _spl_f(0.0)
        m1 = _spl_f(0.0)
        m2 = _spl_f(0.0)
        m3 = _spl_f(0.0)
        bv2 = []
        bi2 = []
        for g in range(G):
            g0 = gt_bc(0, g)
            g1 = gt_bc(1, g)
            g2 = gt_bc(2, g)
            g3 = gt_bc(3, g)
            ga = gt_bc(4, g)
            wi = jnp.maximum(jnp.minimum(g2, px2) - jnp.maximum(g0, px1), 0.0)
            hi = jnp.maximum(jnp.minimum(g3, py2) - jnp.maximum(g1, py1), 0.0)
            inter = wi * hi
            iou = inter / (ga + parea - inter)
            m = iou > btv
            btv = jnp.where(m, iou, btv)
            m0 = jnp.where(m, g0, m0)
            m1 = jnp.where(m, g1, m1)
            m2 = jnp.where(m, g2, m2)
            m3 = jnp.where(m, g3, m3)
            if g < 8:
                mg = iou > bv8[g]
                bv2.append(jnp.where(mg, iou, bv8[g]))
                bi2.append(jnp.where(mg, pidx, bi8[g]))
        mbuf[0, pl.ds(off, 16)] = btv
        mbuf[1, pl.ds(off, 16)] = m0
        mbuf[2, pl.ds(off, 16)] = m1
        mbuf[3, pl.ds(off, 16)] = m2
        mbuf[4, pl.ds(off, 16)] = m3
        return tuple(bv2), tuple(bi2)

    def a1c_body(ci, vi, st):
        a_sl1, a_ce, a_np = st
        off = vi * 16
        cx = buf_pb[0, pl.ds(off, 16)]
        cy = buf_pb[1, pl.ds(off, 16)]
        w = buf_pb[2, pl.ds(off, 16)]
        h = buf_pb[3, pl.ds(off, 16)]
        btv = mbuf[0, pl.ds(off, 16)]
        m0 = mbuf[1, pl.ds(off, 16)]
        m1 = mbuf[2, pl.ds(off, 16)]
        m2 = mbuf[3, pl.ds(off, 16)]
        m3 = mbuf[4, pl.ds(off, 16)]
        pos = btv >= THR
        posf = jnp.where(pos, 1.0, 0.0)
        ecx = ((m0 + m2) / 2 - cx) / (V0 * w)
        ecy = ((m1 + m3) / 2 - cy) / (V0 * h)
        ew = _ln(jnp.maximum((m2 - m0) / w, 1e-8)) / V1
        eh = _ln(jnp.maximum((m3 - m1) / h, 1e-8)) / V1
        s = _spl_f(0.0)
        for comp, enc in ((0, ecx), (1, ecy), (2, ew), (3, eh)):
            d = buf_loc[comp, pl.ds(off, 16)] - enc
            ad = jnp.abs(d)
            s = s + jnp.where(ad < 1.0, 0.5 * d * d, ad - 0.5)
        a_sl1 = a_sl1 + s * posf
        x0 = buf_cls[0, pl.ds(off, 16)]
        x1 = buf_cls[1, pl.ds(off, 16)]
        mx = jnp.maximum(x0, x1)
        z = jnp.exp(-jnp.abs(x0 - x1))
        lse = mx + _ln_m(1.0 + z)
        ce1 = lse - x1
        ce0 = lse - x0
        a_ce = a_ce + jnp.where(pos, ce1, 0.0)
        a_np = a_np + jnp.where(pos, 1, 0)
        lc[pl.ds(ci * CHUNK + off, 16)] = jnp.where(pos, _spl_f(-0.0), ce0)
        return a_sl1, a_ce, a_np

    def a2_body(ci, gbase, vi, st):
        bv8, bi8 = st
        off = vi * 16
        pidx = _spl_i(ci * CHUNK) + _spl_i(off) + it16
        cx = buf_pb[0, pl.ds(off, 16)]
        cy = buf_pb[1, pl.ds(off, 16)]
        w = buf_pb[2, pl.ds(off, 16)]
        h = buf_pb[3, pl.ds(off, 16)]
        px1 = cx - w / 2
        py1 = cy - h / 2
        px2 = cx + w / 2
        py2 = cy + h / 2
        parea = (px2 - px1) * (py2 - py1)
        bv2 = []
        bi2 = []
        for j in range(8):
            g = gbase + j
            g0 = gt_bc(0, g)
            g1 = gt_bc(1, g)
            g2 = gt_bc(2, g)
            g3 = gt_bc(3, g)
            ga = gt_bc(4, g)
            wi = jnp.maximum(jnp.minimum(g2, px2) - jnp.maximum(g0, px1), 0.0)
            hi = jnp.maximum(jnp.minimum(g3, py2) - jnp.maximum(g1, py1), 0.0)
            inter = wi * hi
            iou = inter / (ga + parea - inter)
            mg = iou > bv8[j]
            bv2.append(jnp.where(mg, iou, bv8[j]))
            bi2.append(jnp.where(mg, pidx, bi8[j]))
        return tuple(bv2), tuple(bi2)

    def chunk_body(ci, st):
        bv, bi, a_sl1, a_ce, a_np = st
        pltpu.sync_copy(pbt.at[:, pl.ds(ci * CHUNK, CHUNK)], buf_pb)
        pltpu.sync_copy(loct.at[wid, :, pl.ds(ci * CHUNK, CHUNK)], buf_loc)
        pltpu.sync_copy(clst.at[wid, :, pl.ds(ci * CHUNK, CHUNK)], buf_cls)
        lo = lax.fori_loop(0, NVEC, functools.partial(a1_body, ci),
                           (bv[:8], bi[:8]), unroll=2)
        a_sl1, a_ce, a_np = lax.fori_loop(
            0, NVEC, functools.partial(a1c_body, ci),
            (a_sl1, a_ce, a_np), unroll=2)
        hi = lax.fori_loop(0, NVEC, functools.partial(a2_body, ci, 8),
                           (bv[8:], bi[8:]), unroll=2)
        return (lo[0] + hi[0], lo[1] + hi[1], a_sl1, a_ce, a_np)

    init = (tuple(_spl_f(-1.0) for _ in range(G)),
            tuple(_spl_i(0) for _ in range(G)),
            _spl_f(0.0), _spl_f(0.0), _spl_i(0))
    bv, bi, a_sl1, a_ce, a_np = lax.fori_loop(0, NCHUNK, chunk_body, init, unroll=False)

    bpi = _spl_i(0)
    pgs = []
    for g in range(G):
        gmax = jnp.max(bv[g])
        cand = jnp.where(bv[g] == gmax, bi[g], P)
        pg = jnp.min(cand)
        pgs.append(pg)
        bpi = jnp.where(it16 == g, pg, bpi)
    loser = it16 < 0
    for j in range(G):
        loser = loser | ((it16 < j) & (bpi == pgs[j]))
    winner = jnp.logical_not(loser)
    def fetch(hbm_flat, base):
        idx_ref[...] = bpi + _spl_i(base)
        pltpu.async_copy(hbm_flat.at[idx_ref], gvec, sem).wait()
        return gvec[...]

    pcx = fetch(pbf, 0)
    pcy = fetch(pbf, P)
    pw = fetch(pbf, 2 * P)
    ph = fetch(pbf, 3 * P)
    lp = [fetch(locf, (wid * 4 + c) * P) for c in range(4)]
    bx0 = fetch(clsf, wid * 2 * P)
    bx1 = fetch(clsf, (wid * 2 + 1) * P)
    px1 = pcx - pw / 2
    py1 = pcy - ph / 2
    px2 = pcx + pw / 2
    py2 = pcy + ph / 2
    parea = (px2 - px1) * (py2 - py1)
    obtv = _spl_f(-1.0)
    om0 = _spl_f(0.0)
    om1 = _spl_f(0.0)
    om2 = _spl_f(0.0)
    om3 = _spl_f(0.0)
    for g in range(G):
        g0 = gt_bc(0, g)
        g1 = gt_bc(1, g)
        g2 = gt_bc(2, g)
        g3 = gt_bc(3, g)
        ga = gt_bc(4, g)
        wi = jnp.maximum(jnp.minimum(g2, px2) - jnp.maximum(g0, px1), 0.0)
        hi = jnp.maximum(jnp.minimum(g3, py2) - jnp.maximum(g1, py1), 0.0)
        inter = wi * hi
        iou = inter / (ga + parea - inter)
        m = iou > obtv
        obtv = jnp.where(m, iou, obtv)
        om0 = jnp.where(m, g0, om0)
        om1 = jnp.where(m, g1, om1)
        om2 = jnp.where(m, g2, om2)
        om3 = jnp.where(m, g3, om3)
    old_pos = obtv >= THR

    def enc_sl1(m0, m1, m2, m3):
        ecx = ((m0 + m2) / 2 - pcx) / (V0 * pw)
        ecy = ((m1 + m3) / 2 - pcy) / (V0 * ph)
        ew = _ln(jnp.maximum((m2 - m0) / pw, 1e-8)) / V1
        eh = _ln(jnp.maximum((m3 - m1) / ph, 1e-8)) / V1
        s = _spl_f(0.0)
        for comp, enc in ((0, ecx), (1, ecy), (2, ew), (3, eh)):
            d = lp[comp] - enc
            ad = jnp.abs(d)
            s = s + jnp.where(ad < 1.0, 0.5 * d * d, ad - 0.5)
        return s

    sl1_new = enc_sl1(c0r, c1r, c2r, c3r)
    sl1_old = enc_sl1(om0, om1, om2, om3)
    z = jnp.exp(-jnp.abs(bx0 - bx1))
    lse = jnp.maximum(bx0, bx1) + _ln_m(1.0 + z)
    ce1 = lse - bx1
    newpos = winner & jnp.logical_not(old_pos)
    a_sl1 = a_sl1 + jnp.where(winner, sl1_new - jnp.where(old_pos, sl1_old, 0.0), 0.0)
    a_ce = a_ce + jnp.where(newpos, ce1, 0.0)
    a_np = a_np + jnp.where(newpos, 1, 0)
    plsc.store_scatter(lc, [bpi], _spl_f(-0.0), mask=winner)

    r_sl1 = jnp.sum(a_sl1)
    r_ce = jnp.sum(a_ce)
    r_np = jnp.sum(a_np)

    k = jnp.minimum(NEGPOS * r_np, P - 1)

    def zero_body(j, _):
        hist_cnt[pl.ds(j * 16, 16)] = _spl_i(0)
        hist_sum[pl.ds(j * 16, 16)] = _spl_f(0.0)
        return 0

    lax.fori_loop(0, NBIN // 16, zero_body, 0, unroll=False)

    def hist_body(j, _):
        v = lc[pl.ds(j * 16, 16)]
        mb = lax.bitcast_convert_type(v, i32) & 0x7FFFFFFF
        bn = mb >> 19
        plsc.addupdate_scatter(hist_cnt, [bn], _spl_i(1))
        plsc.addupdate_scatter(hist_sum, [bn], jnp.abs(v))
        return 0

    lax.fori_loop(0, P // 16, hist_body, 0, unroll=False)

    def scan_body(j, st):
        done, run_c, run_s, beta, cnt_hi, sum_hi = st
        jj = NBIN // 16 - 1 - j
        cv = hist_cnt[pl.ds(jj * 16, 16)]
        sv = hist_sum[pl.ds(jj * 16, 16)]
        s = jnp.sum(cv)
        found = jnp.logical_and(jnp.logical_not(done), run_c + s >= k)
        incl = plsc.cumsum(cv)
        cum_top = run_c + s - incl + cv
        maskl = cum_top >= k
        nb = _popcount(maskl)
        beta_lane = nb - 1
        beta_c = jj * 16 + beta_lane
        cnt_hi_c = run_c + jnp.sum(jnp.where(it16 > beta_lane, cv, 0))
        sum_hi_c = run_s + jnp.sum(jnp.where(it16 > beta_lane, sv, 0.0))
        beta = jnp.where(found, beta_c, beta)
        cnt_hi = jnp.where(found, cnt_hi_c, cnt_hi)
        sum_hi = jnp.where(found, sum_hi_c, sum_hi)
        done2 = jnp.logical_or(done, found)
        run_c = jnp.where(done2, run_c, run_c + s)
        run_s = jnp.where(done2, run_s, run_s + jnp.sum(sv))
        return done2, run_c, run_s, beta, cnt_hi, sum_hi

    _, _, _, beta, cnt_hi, sum_hi = lax.fori_loop(
        0, NBIN // 16, scan_body,
        (jnp.bool_(False), jnp.int32(0), jnp.float32(0.0),
         jnp.int32(0), jnp.int32(0), jnp.float32(0.0)), unroll=False)

    def subzero_body(j, _):
        subset[pl.ds(j * 16, 16)] = _spl_i(0)
        return 0

    lax.fori_loop(0, (SUBSET + 16) // 16, subzero_body, 0, unroll=False)

    def compact_body(j, off):
        v = lc[pl.ds(j * 16, 16)]
        mb = lax.bitcast_convert_type(v, i32) & 0x7FFFFFFF
        m = jnp.logical_and(mb >> 19 == beta, off < SUBSET)
        plsc.store_compressed(subset.at[pl.ds(off, 16)], mb, mask=m)
        return off + _popcount(m)

    s_cnt = lax.fori_loop(0, P // 16, compact_body, jnp.int32(0), unroll=False)
    nvec = (s_cnt + 15) // 16
    r = k - cnt_hi

    def count_ge(u):
        def cbody(i, acc):
            sv = subset[pl.ds(i * 16, 16)]
            valid = (it16 + i * 16) < s_cnt
            mm = jnp.logical_and(sv >= u, valid)
            return acc + plsc.all_reduce_population_count(mm)
        return jnp.max(lax.fori_loop(0, nvec, cbody, _spl_i(0)))

    lo = beta << 19
    hi = lo | 0x7FFFF
    for _ in range(19):
        mid = lo + ((hi - lo + 1) >> 1)
        ge = count_ge(mid) >= r
        lo = jnp.where(ge, mid, lo)
        hi = jnp.where(ge, hi, mid - 1)
    tbits = lo
    tval = lax.bitcast_convert_type(tbits, f32)

    def above_body(i, st):
        c, sacc = st
        sv = subset[pl.ds(i * 16, 16)]
        valid = (it16 + i * 16) < s_cnt
        mm = jnp.logical_and(sv > tbits, valid)
        c = c + plsc.all_reduce_population_count(mm)
        sacc = sacc + jnp.where(mm, lax.bitcast_convert_type(sv, f32), 0.0)
        return c, sacc

    c2v, s2v = lax.fori_loop(0, nvec, above_body, (_spl_i(0), _spl_f(0.0)))
    c2 = jnp.max(c2v)
    s2 = jnp.sum(s2v)
    cnt_gt = cnt_hi + c2
    sum_gt = sum_hi + s2
    topk = sum_gt + (k - cnt_gt).astype(f32) * tval
    numer = r_ce + topk

    def edge_fn(_):
        need = k - cnt_gt

        def ebody(j, st):
            run, extra = st
            v = lc[pl.ds(j * 16, 16)]
            bb = lax.bitcast_convert_type(v, i32)
            zm = (bb & 0x7FFFFFFF) == 0
            incl = plsc.cumsum(jnp.where(zm, 1, 0))
            sel = jnp.logical_and(zm, (run + incl) <= need)
            extra = extra + _popcount(jnp.logical_and(sel, bb == 0))
            return run + _popcount(zm), extra

        _, extra = lax.fori_loop(0, P // 16, ebody, (jnp.int32(0), jnp.int32(0)))
        return r_np + cnt_gt + extra

    mask_cnt = lax.cond(tbits == 0, edge_fn, lambda _: r_np + k, 0)

    o = jnp.where(it16 == 0, r_sl1,
                  jnp.where(it16 == 1, r_np.astype(f32),
                            jnp.where(it16 == 2, numer,
                                      jnp.where(it16 == 3, mask_cnt.astype(f32), 0.0))))
    outv[...] = o
    pltpu.sync_copy(outv, out.at[wid])


@jax.jit
def kernel(loc_preds, cls_preds, priorbox, targets):
    pbt = priorbox.T
    loct = jnp.transpose(loc_preds, (0, 2, 1))
    clst = jnp.transpose(cls_preds, (0, 2, 1))
    pbf = pbt.reshape(4 * P)
    locf = loct.reshape(B * 4 * P)
    clsf = clst.reshape(B * 2 * P)
    tgf = targets.reshape(B, G * 5)
    mesh = plsc.VectorSubcoreMesh(core_axis_name="c", subcore_axis_name="s",
                                  num_cores=NC, num_subcores=NS)
    out = pl.kernel(
        _sc_body,
        out_type=jax.ShapeDtypeStruct((B, 16), f32),
        mesh=mesh,
        compiler_params=pltpu.CompilerParams(needs_layout_passes=False),
        scratch_types=[
            pltpu.VMEM((4, CHUNK), f32),
            pltpu.VMEM((4, CHUNK), f32),
            pltpu.VMEM((2, CHUNK), f32),
            pltpu.VMEM((P,), f32),
            pltpu.VMEM((G * 5,), f32),
            pltpu.VMEM((5 * G * 16,), f32),
            pltpu.VMEM((NBIN,), i32),
            pltpu.VMEM((NBIN,), f32),
            pltpu.VMEM((SUBSET + 16,), i32),
            pltpu.VMEM((16,), f32),
            pltpu.VMEM((16,), i32),
            pltpu.VMEM((16,), f32),
            pltpu.VMEM((5, CHUNK), f32),
            pltpu.SemaphoreType.DMA,
        ],
    )(pbt, loct, clst, tgf, pbf, locf, clsf)
    tot = jnp.sum(out, axis=0)
    tot_sl1, tot_pos, tot_num, tot_mask = tot[0], tot[1], tot[2], tot[3]
    loss_loc = tot_sl1 / jnp.maximum(tot_pos * 4.0, 1.0)
    loss_cls = tot_num / jnp.maximum(tot_mask, 1.0)
    loss = (loss_cls + loss_loc) / jnp.maximum(tot_pos, 1.0)
    return (loss, loss_loc, loss_cls)


# speedup vs baseline: 68.3290x; 1.1051x over previous
import functools

import jax
import jax.numpy as jnp
from jax import lax
from jax.experimental import pallas as pl
from jax.experimental.pallas import tpu as pltpu
from jax.experimental.pallas import tpu_sc as plsc

B = 32
P = 32768
G = 16
NC = 2
NS = 16
CHUNK = 2048
NCHUNK = P // CHUNK
NVEC = CHUNK // 16
THR = 0.35
V0, V1 = 0.1, 0.2
NEGPOS = 3
NBIN = 4096
SUBSET = 8192
LN2 = 0.6931471805599453

f32 = jnp.float32
i32 = jnp.int32


def _i16():
    return lax.iota(i32, 16)


def _spl_f(x):
    return jnp.full((16,), x, f32)


def _spl_i(x):
    return jnp.full((16,), x, i32)


def _ln_m(x):
    s = (x - 1.0) / (x + 1.0)
    s2 = s * s
    return s * (2.0 + s2 * (2.0 / 3.0 + s2 * (2.0 / 5.0 + s2 * (2.0 / 7.0 + s2 * (2.0 / 9.0)))))


def _ln(x):
    b = lax.bitcast_convert_type(x, i32)
    e = (b >> 23) - 127
    m = lax.bitcast_convert_type((b & 0x7FFFFF) | 0x3F800000, f32)
    return e.astype(f32) * LN2 + _ln_m(m)


def _popcount(mask):
    return jnp.max(plsc.all_reduce_population_count(mask))


def _sc_body(pbt, loct, clst, tgt, pbf, locf, clsf, out,
             buf_pb, buf_loc, buf_cls, lc, tgv, gtc, hist_cnt, hist_sum,
             subset, gvec, idx_ref, outv, mbuf, sem):
    wid = lax.axis_index("s") * NC + lax.axis_index("c")
    it16 = _i16()

    pltpu.sync_copy(tgt.at[wid], tgv)
    tw = [tgv[pl.ds(o * 16, 16)] for o in range(5)]

    def tsc(j):
        return tw[j // 16][j % 16]

    c0r = _spl_f(0.0)
    c1r = _spl_f(0.0)
    c2r = _spl_f(0.0)
    c3r = _spl_f(0.0)
    for g in range(G):
        t0 = tsc(5 * g)
        t1 = tsc(5 * g + 1)
        t2 = tsc(5 * g + 2)
        t3 = tsc(5 * g + 3)
        mg = it16 == g
        c0r = jnp.where(mg, t1, c0r)
        c1r = jnp.where(mg, t0, c1r)
        c2r = jnp.where(mg, t3, c2r)
        c3r = jnp.where(mg, t2, c3r)
        ag = (t3 - t1) * (t2 - t0)
        gtc[pl.ds((0 * G + g) * 16, 16)] = jnp.full((16,), t1, f32)
        gtc[pl.ds((1 * G + g) * 16, 16)] = jnp.full((16,), t0, f32)
        gtc[pl.ds((2 * G + g) * 16, 16)] = jnp.full((16,), t3, f32)
        gtc[pl.ds((3 * G + g) * 16, 16)] = jnp.full((16,), t2, f32)
        gtc[pl.ds((4 * G + g) * 16, 16)] = jnp.full((16,), ag, f32)
    def gt_bc(comp, g):
        return gtc[pl.ds((comp * G + g) * 16, 16)]

    def a1_body(ci, vi, st):
        bv8, bi8 = st
        off = vi * 16
        pidx = _spl_i(ci * CHUNK) + _spl_i(off) + it16
        cx = buf_pb[0, pl.ds(off, 16)]
        cy = buf_pb[1, pl.ds(off, 16)]
        w = buf_pb[2, pl.ds(off, 16)]
        h = buf_pb[3, pl.ds(off, 16)]
        px1 = cx - w / 2
        py1 = cy - h / 2
        px2 = cx + w / 2
        py2 = cy + h / 2
        parea = (px2 - px1) * (py2 - py1)
        btv = _spl_f(-1.0)
        m0 = _spl_f(0.0)
        m1 = _spl_f(0.0)
        m2 = _spl_f(0.0)
        m3 = _spl_f(0.0)
        bv2 = []
        bi2 = []
        for g in range(G):
            g0 = gt_bc(0, g)
            g1 = gt_bc(1, g)
            g2 = gt_bc(2, g)
            g3 = gt_bc(3, g)
            ga = gt_bc(4, g)
            wi = jnp.maximum(jnp.minimum(g2, px2) - jnp.maximum(g0, px1), 0.0)
            hi = jnp.maximum(jnp.minimum(g3, py2) - jnp.maximum(g1, py1), 0.0)
            inter = wi * hi
            iou = inter / (ga + parea - inter)
            m = iou > btv
            btv = jnp.where(m, iou, btv)
            m0 = jnp.where(m, g0, m0)
            m1 = jnp.where(m, g1, m1)
            m2 = jnp.where(m, g2, m2)
            m3 = jnp.where(m, g3, m3)
            if True:
                mg = iou > bv8[g]
                bv2.append(jnp.where(mg, iou, bv8[g]))
                bi2.append(jnp.where(mg, pidx, bi8[g]))
        mbuf[0, pl.ds(off, 16)] = btv
        mbuf[1, pl.ds(off, 16)] = m0
        mbuf[2, pl.ds(off, 16)] = m1
        mbuf[3, pl.ds(off, 16)] = m2
        mbuf[4, pl.ds(off, 16)] = m3
        return tuple(bv2), tuple(bi2)

    def a1c_body(ci, vi, st):
        a_sl1, a_ce, a_np = st
        off = vi * 16
        cx = buf_pb[0, pl.ds(off, 16)]
        cy = buf_pb[1, pl.ds(off, 16)]
        w = buf_pb[2, pl.ds(off, 16)]
        h = buf_pb[3, pl.ds(off, 16)]
        btv = mbuf[0, pl.ds(off, 16)]
        m0 = mbuf[1, pl.ds(off, 16)]
        m1 = mbuf[2, pl.ds(off, 16)]
        m2 = mbuf[3, pl.ds(off, 16)]
        m3 = mbuf[4, pl.ds(off, 16)]
        pos = btv >= THR
        posf = jnp.where(pos, 1.0, 0.0)
        ecx = ((m0 + m2) / 2 - cx) / (V0 * w)
        ecy = ((m1 + m3) / 2 - cy) / (V0 * h)
        ew = _ln(jnp.maximum((m2 - m0) / w, 1e-8)) / V1
        eh = _ln(jnp.maximum((m3 - m1) / h, 1e-8)) / V1
        s = _spl_f(0.0)
        for comp, enc in ((0, ecx), (1, ecy), (2, ew), (3, eh)):
            d = buf_loc[comp, pl.ds(off, 16)] - enc
            ad = jnp.abs(d)
            s = s + jnp.where(ad < 1.0, 0.5 * d * d, ad - 0.5)
        a_sl1 = a_sl1 + s * posf
        x0 = buf_cls[0, pl.ds(off, 16)]
        x1 = buf_cls[1, pl.ds(off, 16)]
        mx = jnp.maximum(x0, x1)
        z = jnp.exp(-jnp.abs(x0 - x1))
        lse = mx + _ln_m(1.0 + z)
        ce1 = lse - x1
        ce0 = lse - x0
        a_ce = a_ce + jnp.where(pos, ce1, 0.0)
        a_np = a_np + jnp.where(pos, 1, 0)
        lc[pl.ds(ci * CHUNK + off, 16)] = jnp.where(pos, _spl_f(-0.0), ce0)
        return a_sl1, a_ce, a_np

    def a2_body(ci, gbase, vi, st):
        bv8, bi8 = st
        off = vi * 16
        pidx = _spl_i(ci * CHUNK) + _spl_i(off) + it16
        cx = buf_pb[0, pl.ds(off, 16)]
        cy = buf_pb[1, pl.ds(off, 16)]
        w = buf_pb[2, pl.ds(off, 16)]
        h = buf_pb[3, pl.ds(off, 16)]
        px1 = cx - w / 2
        py1 = cy - h / 2
        px2 = cx + w / 2
        py2 = cy + h / 2
        parea = (px2 - px1) * (py2 - py1)
        bv2 = []
        bi2 = []
        for j in range(8):
            g = gbase + j
            g0 = gt_bc(0, g)
            g1 = gt_bc(1, g)
            g2 = gt_bc(2, g)
            g3 = gt_bc(3, g)
            ga = gt_bc(4, g)
            wi = jnp.maximum(jnp.minimum(g2, px2) - jnp.maximum(g0, px1), 0.0)
            hi = jnp.maximum(jnp.minimum(g3, py2) - jnp.maximum(g1, py1), 0.0)
            inter = wi * hi
            iou = inter / (ga + parea - inter)
            mg = iou > bv8[j]
            bv2.append(jnp.where(mg, iou, bv8[j]))
            bi2.append(jnp.where(mg, pidx, bi8[j]))
        return tuple(bv2), tuple(bi2)

    def chunk_body(ci, st):
        bv, bi, a_sl1, a_ce, a_np = st
        pltpu.sync_copy(pbt.at[:, pl.ds(ci * CHUNK, CHUNK)], buf_pb)
        pltpu.sync_copy(loct.at[wid, :, pl.ds(ci * CHUNK, CHUNK)], buf_loc)
        pltpu.sync_copy(clst.at[wid, :, pl.ds(ci * CHUNK, CHUNK)], buf_cls)
        bv, bi = lax.fori_loop(0, NVEC, functools.partial(a1_body, ci),
                               (bv, bi), unroll=2)
        a_sl1, a_ce, a_np = lax.fori_loop(
            0, NVEC, functools.partial(a1c_body, ci),
            (a_sl1, a_ce, a_np), unroll=2)
        return (bv, bi, a_sl1, a_ce, a_np)

    init = (tuple(_spl_f(-1.0) for _ in range(G)),
            tuple(_spl_i(0) for _ in range(G)),
            _spl_f(0.0), _spl_f(0.0), _spl_i(0))
    bv, bi, a_sl1, a_ce, a_np = lax.fori_loop(0, NCHUNK, chunk_body, init, unroll=False)

    bpi = _spl_i(0)
    pgs = []
    for g in range(G):
        gmax = jnp.max(bv[g])
        cand = jnp.where(bv[g] == gmax, bi[g], P)
        pg = jnp.min(cand)
        pgs.append(pg)
        bpi = jnp.where(it16 == g, pg, bpi)
    loser = it16 < 0
    for j in range(G):
        loser = loser | ((it16 < j) & (bpi == pgs[j]))
    winner = jnp.logical_not(loser)
    def fetch(hbm_flat, base):
        idx_ref[...] = bpi + _spl_i(base)
        pltpu.async_copy(hbm_flat.at[idx_ref], gvec, sem).wait()
        return gvec[...]

    pcx = fetch(pbf, 0)
    pcy = fetch(pbf, P)
    pw = fetch(pbf, 2 * P)
    ph = fetch(pbf, 3 * P)
    lp = [fetch(locf, (wid * 4 + c) * P) for c in range(4)]
    bx0 = fetch(clsf, wid * 2 * P)
    bx1 = fetch(clsf, (wid * 2 + 1) * P)
    px1 = pcx - pw / 2
    py1 = pcy - ph / 2
    px2 = pcx + pw / 2
    py2 = pcy + ph / 2
    parea = (px2 - px1) * (py2 - py1)
    obtv = _spl_f(-1.0)
    om0 = _spl_f(0.0)
    om1 = _spl_f(0.0)
    om2 = _spl_f(0.0)
    om3 = _spl_f(0.0)
    for g in range(G):
        g0 = gt_bc(0, g)
        g1 = gt_bc(1, g)
        g2 = gt_bc(2, g)
        g3 = gt_bc(3, g)
        ga = gt_bc(4, g)
        wi = jnp.maximum(jnp.minimum(g2, px2) - jnp.maximum(g0, px1), 0.0)
        hi = jnp.maximum(jnp.minimum(g3, py2) - jnp.maximum(g1, py1), 0.0)
        inter = wi * hi
        iou = inter / (ga + parea - inter)
        m = iou > obtv
        obtv = jnp.where(m, iou, obtv)
        om0 = jnp.where(m, g0, om0)
        om1 = jnp.where(m, g1, om1)
        om2 = jnp.where(m, g2, om2)
        om3 = jnp.where(m, g3, om3)
    old_pos = obtv >= THR

    def enc_sl1(m0, m1, m2, m3):
        ecx = ((m0 + m2) / 2 - pcx) / (V0 * pw)
        ecy = ((m1 + m3) / 2 - pcy) / (V0 * ph)
        ew = _ln(jnp.maximum((m2 - m0) / pw, 1e-8)) / V1
        eh = _ln(jnp.maximum((m3 - m1) / ph, 1e-8)) / V1
        s = _spl_f(0.0)
        for comp, enc in ((0, ecx), (1, ecy), (2, ew), (3, eh)):
            d = lp[comp] - enc
            ad = jnp.abs(d)
            s = s + jnp.where(ad < 1.0, 0.5 * d * d, ad - 0.5)
        return s

    sl1_new = enc_sl1(c0r, c1r, c2r, c3r)
    sl1_old = enc_sl1(om0, om1, om2, om3)
    z = jnp.exp(-jnp.abs(bx0 - bx1))
    lse = jnp.maximum(bx0, bx1) + _ln_m(1.0 + z)
    ce1 = lse - bx1
    newpos = winner & jnp.logical_not(old_pos)
    a_sl1 = a_sl1 + jnp.where(winner, sl1_new - jnp.where(old_pos, sl1_old, 0.0), 0.0)
    a_ce = a_ce + jnp.where(newpos, ce1, 0.0)
    a_np = a_np + jnp.where(newpos, 1, 0)
    plsc.store_scatter(lc, [bpi], _spl_f(-0.0), mask=winner)

    r_sl1 = jnp.sum(a_sl1)
    r_ce = jnp.sum(a_ce)
    r_np = jnp.sum(a_np)

    k = jnp.minimum(NEGPOS * r_np, P - 1)

    def zero_body(j, _):
        hist_cnt[pl.ds(j * 16, 16)] = _spl_i(0)
        hist_sum[pl.ds(j * 16, 16)] = _spl_f(0.0)
        return 0

    lax.fori_loop(0, NBIN // 16, zero_body, 0, unroll=False)

    def hist_body(j, _):
        v = lc[pl.ds(j * 16, 16)]
        mb = lax.bitcast_convert_type(v, i32) & 0x7FFFFFFF
        bn = mb >> 19
        plsc.addupdate_scatter(hist_cnt, [bn], _spl_i(1))
        plsc.addupdate_scatter(hist_sum, [bn], jnp.abs(v))
        return 0

    lax.fori_loop(0, P // 16, hist_body, 0, unroll=False)

    def scan_body(j, st):
        done, run_c, run_s, beta, cnt_hi, sum_hi = st
        jj = NBIN // 16 - 1 - j
        cv = hist_cnt[pl.ds(jj * 16, 16)]
        sv = hist_sum[pl.ds(jj * 16, 16)]
        s = jnp.sum(cv)
        found = jnp.logical_and(jnp.logical_not(done), run_c + s >= k)
        incl = plsc.cumsum(cv)
        cum_top = run_c + s - incl + cv
        maskl = cum_top >= k
        nb = _popcount(maskl)
        beta_lane = nb - 1
        beta_c = jj * 16 + beta_lane
        cnt_hi_c = run_c + jnp.sum(jnp.where(it16 > beta_lane, cv, 0))
        sum_hi_c = run_s + jnp.sum(jnp.where(it16 > beta_lane, sv, 0.0))
        beta = jnp.where(found, beta_c, beta)
        cnt_hi = jnp.where(found, cnt_hi_c, cnt_hi)
        sum_hi = jnp.where(found, sum_hi_c, sum_hi)
        done2 = jnp.logical_or(done, found)
        run_c = jnp.where(done2, run_c, run_c + s)
        run_s = jnp.where(done2, run_s, run_s + jnp.sum(sv))
        return done2, run_c, run_s, beta, cnt_hi, sum_hi

    _, _, _, beta, cnt_hi, sum_hi = lax.fori_loop(
        0, NBIN // 16, scan_body,
        (jnp.bool_(False), jnp.int32(0), jnp.float32(0.0),
         jnp.int32(0), jnp.int32(0), jnp.float32(0.0)), unroll=False)

    def subzero_body(j, _):
        subset[pl.ds(j * 16, 16)] = _spl_i(0)
        return 0

    lax.fori_loop(0, (SUBSET + 16) // 16, subzero_body, 0, unroll=False)

    def compact_body(j, off):
        v = lc[pl.ds(j * 16, 16)]
        mb = lax.bitcast_convert_type(v, i32) & 0x7FFFFFFF
        m = jnp.logical_and(mb >> 19 == beta, off < SUBSET)
        plsc.store_compressed(subset.at[pl.ds(off, 16)], mb, mask=m)
        return off + _popcount(m)

    s_cnt = lax.fori_loop(0, P // 16, compact_body, jnp.int32(0), unroll=False)
    nvec = (s_cnt + 15) // 16
    r = k - cnt_hi

    def count_ge(u):
        def cbody(i, acc):
            sv = subset[pl.ds(i * 16, 16)]
            valid = (it16 + i * 16) < s_cnt
            mm = jnp.logical_and(sv >= u, valid)
            return acc + plsc.all_reduce_population_count(mm)
        return jnp.max(lax.fori_loop(0, nvec, cbody, _spl_i(0)))

    lo = beta << 19
    hi = lo | 0x7FFFF
    for _ in range(19):
        mid = lo + ((hi - lo + 1) >> 1)
        ge = count_ge(mid) >= r
        lo = jnp.where(ge, mid, lo)
        hi = jnp.where(ge, hi, mid - 1)
    tbits = lo
    tval = lax.bitcast_convert_type(tbits, f32)

    def above_body(i, st):
        c, sacc = st
        sv = subset[pl.ds(i * 16, 16)]
        valid = (it16 + i * 16) < s_cnt
        mm = jnp.logical_and(sv > tbits, valid)
        c = c + plsc.all_reduce_population_count(mm)
        sacc = sacc + jnp.where(mm, lax.bitcast_convert_type(sv, f32), 0.0)
        return c, sacc

    c2v, s2v = lax.fori_loop(0, nvec, above_body, (_spl_i(0), _spl_f(0.0)))
    c2 = jnp.max(c2v)
    s2 = jnp.sum(s2v)
    cnt_gt = cnt_hi + c2
    sum_gt = sum_hi + s2
    topk = sum_gt + (k - cnt_gt).astype(f32) * tval
    numer = r_ce + topk

    def edge_fn(_):
        need = k - cnt_gt

        def ebody(j, st):
            run, extra = st
            v = lc[pl.ds(j * 16, 16)]
            bb = lax.bitcast_convert_type(v, i32)
            zm = (bb & 0x7FFFFFFF) == 0
            incl = plsc.cumsum(jnp.where(zm, 1, 0))
            sel = jnp.logical_and(zm, (run + incl) <= need)
            extra = extra + _popcount(jnp.logical_and(sel, bb == 0))
            return run + _popcount(zm), extra

        _, extra = lax.fori_loop(0, P // 16, ebody, (jnp.int32(0), jnp.int32(0)))
        return r_np + cnt_gt + extra

    mask_cnt = lax.cond(tbits == 0, edge_fn, lambda _: r_np + k, 0)

    o = jnp.where(it16 == 0, r_sl1,
                  jnp.where(it16 == 1, r_np.astype(f32),
                            jnp.where(it16 == 2, numer,
                                      jnp.where(it16 == 3, mask_cnt.astype(f32), 0.0))))
    outv[...] = o
    pltpu.sync_copy(outv, out.at[wid])


@jax.jit
def kernel(loc_preds, cls_preds, priorbox, targets):
    pbt = priorbox.T
    loct = jnp.transpose(loc_preds, (0, 2, 1))
    clst = jnp.transpose(cls_preds, (0, 2, 1))
    pbf = pbt.reshape(4 * P)
    locf = loct.reshape(B * 4 * P)
    clsf = clst.reshape(B * 2 * P)
    tgf = targets.reshape(B, G * 5)
    mesh = plsc.VectorSubcoreMesh(core_axis_name="c", subcore_axis_name="s",
                                  num_cores=NC, num_subcores=NS)
    out = pl.kernel(
        _sc_body,
        out_type=jax.ShapeDtypeStruct((B, 16), f32),
        mesh=mesh,
        compiler_params=pltpu.CompilerParams(needs_layout_passes=False),
        scratch_types=[
            pltpu.VMEM((4, CHUNK), f32),
            pltpu.VMEM((4, CHUNK), f32),
            pltpu.VMEM((2, CHUNK), f32),
            pltpu.VMEM((P,), f32),
            pltpu.VMEM((G * 5,), f32),
            pltpu.VMEM((5 * G * 16,), f32),
            pltpu.VMEM((NBIN,), i32),
            pltpu.VMEM((NBIN,), f32),
            pltpu.VMEM((SUBSET + 16,), i32),
            pltpu.VMEM((16,), f32),
            pltpu.VMEM((16,), i32),
            pltpu.VMEM((16,), f32),
            pltpu.VMEM((5, CHUNK), f32),
            pltpu.SemaphoreType.DMA,
        ],
    )(pbt, loct, clst, tgf, pbf, locf, clsf)
    tot = jnp.sum(out, axis=0)
    tot_sl1, tot_pos, tot_num, tot_mask = tot[0], tot[1], tot[2], tot[3]
    loss_loc = tot_sl1 / jnp.maximum(tot_pos * 4.0, 1.0)
    loss_cls = tot_num / jnp.maximum(tot_mask, 1.0)
    loss = (loss_cls + loss_loc) / jnp.maximum(tot_pos, 1.0)
    return (loss, loss_loc, loss_cls)


# speedup vs baseline: 70.1405x; 1.0265x over previous
import functools

import jax
import jax.numpy as jnp
from jax import lax
from jax.experimental import pallas as pl
from jax.experimental.pallas import tpu as pltpu
from jax.experimental.pallas import tpu_sc as plsc

B = 32
P = 32768
G = 16
NC = 2
NS = 16
CHUNK = 4096
NCHUNK = P // CHUNK
NVEC = CHUNK // 16
THR = 0.35
V0, V1 = 0.1, 0.2
NEGPOS = 3
NBIN = 4096
SUBSET = 8192
LN2 = 0.6931471805599453

f32 = jnp.float32
i32 = jnp.int32


def _i16():
    return lax.iota(i32, 16)


def _spl_f(x):
    return jnp.full((16,), x, f32)


def _spl_i(x):
    return jnp.full((16,), x, i32)


def _ln_m(x):
    s = (x - 1.0) / (x + 1.0)
    s2 = s * s
    return s * (2.0 + s2 * (2.0 / 3.0 + s2 * (2.0 / 5.0 + s2 * (2.0 / 7.0 + s2 * (2.0 / 9.0)))))


def _ln(x):
    b = lax.bitcast_convert_type(x, i32)
    e = (b >> 23) - 127
    m = lax.bitcast_convert_type((b & 0x7FFFFF) | 0x3F800000, f32)
    return e.astype(f32) * LN2 + _ln_m(m)


def _popcount(mask):
    return jnp.max(plsc.all_reduce_population_count(mask))


def _sc_body(pbt, loct, clst, tgt, pbf, locf, clsf, out,
             buf_pb, buf_loc, buf_cls, lc, tgv, gtc, hist_cnt, hist_sum,
             subset, gvec, idx_ref, outv, mbuf, sem):
    wid = lax.axis_index("s") * NC + lax.axis_index("c")
    it16 = _i16()

    pltpu.sync_copy(tgt.at[wid], tgv)
    tw = [tgv[pl.ds(o * 16, 16)] for o in range(5)]

    def tsc(j):
        return tw[j // 16][j % 16]

    c0r = _spl_f(0.0)
    c1r = _spl_f(0.0)
    c2r = _spl_f(0.0)
    c3r = _spl_f(0.0)
    for g in range(G):
        t0 = tsc(5 * g)
        t1 = tsc(5 * g + 1)
        t2 = tsc(5 * g + 2)
        t3 = tsc(5 * g + 3)
        mg = it16 == g
        c0r = jnp.where(mg, t1, c0r)
        c1r = jnp.where(mg, t0, c1r)
        c2r = jnp.where(mg, t3, c2r)
        c3r = jnp.where(mg, t2, c3r)
        ag = (t3 - t1) * (t2 - t0)
        gtc[pl.ds((0 * G + g) * 16, 16)] = jnp.full((16,), t1, f32)
        gtc[pl.ds((1 * G + g) * 16, 16)] = jnp.full((16,), t0, f32)
        gtc[pl.ds((2 * G + g) * 16, 16)] = jnp.full((16,), t3, f32)
        gtc[pl.ds((3 * G + g) * 16, 16)] = jnp.full((16,), t2, f32)
        gtc[pl.ds((4 * G + g) * 16, 16)] = jnp.full((16,), ag, f32)
    def gt_bc(comp, g):
        return gtc[pl.ds((comp * G + g) * 16, 16)]

    def a1_body(ci, vi, st):
        bv8, bi8 = st
        off = vi * 16
        pidx = _spl_i(ci * CHUNK) + _spl_i(off) + it16
        cx = buf_pb[0, pl.ds(off, 16)]
        cy = buf_pb[1, pl.ds(off, 16)]
        w = buf_pb[2, pl.ds(off, 16)]
        h = buf_pb[3, pl.ds(off, 16)]
        px1 = cx - w / 2
        py1 = cy - h / 2
        px2 = cx + w / 2
        py2 = cy + h / 2
        parea = (px2 - px1) * (py2 - py1)
        btv = _spl_f(-1.0)
        m0 = _spl_f(0.0)
        m1 = _spl_f(0.0)
        m2 = _spl_f(0.0)
        m3 = _spl_f(0.0)
        bv2 = []
        bi2 = []
        for g in range(G):
            g0 = gt_bc(0, g)
            g1 = gt_bc(1, g)
            g2 = gt_bc(2, g)
            g3 = gt_bc(3, g)
            ga = gt_bc(4, g)
            wi = jnp.maximum(jnp.minimum(g2, px2) - jnp.maximum(g0, px1), 0.0)
            hi = jnp.maximum(jnp.minimum(g3, py2) - jnp.maximum(g1, py1), 0.0)
            inter = wi * hi
            iou = inter / (ga + parea - inter)
            m = iou > btv
            btv = jnp.where(m, iou, btv)
            m0 = jnp.where(m, g0, m0)
            m1 = jnp.where(m, g1, m1)
            m2 = jnp.where(m, g2, m2)
            m3 = jnp.where(m, g3, m3)
            if True:
                mg = iou > bv8[g]
                bv2.append(jnp.where(mg, iou, bv8[g]))
                bi2.append(jnp.where(mg, pidx, bi8[g]))
        mbuf[0, pl.ds(off, 16)] = btv
        mbuf[1, pl.ds(off, 16)] = m0
        mbuf[2, pl.ds(off, 16)] = m1
        mbuf[3, pl.ds(off, 16)] = m2
        mbuf[4, pl.ds(off, 16)] = m3
        return tuple(bv2), tuple(bi2)

    def a1c_body(ci, vi, st):
        a_sl1, a_ce, a_np = st
        off = vi * 16
        cx = buf_pb[0, pl.ds(off, 16)]
        cy = buf_pb[1, pl.ds(off, 16)]
        w = buf_pb[2, pl.ds(off, 16)]
        h = buf_pb[3, pl.ds(off, 16)]
        btv = mbuf[0, pl.ds(off, 16)]
        m0 = mbuf[1, pl.ds(off, 16)]
        m1 = mbuf[2, pl.ds(off, 16)]
        m2 = mbuf[3, pl.ds(off, 16)]
        m3 = mbuf[4, pl.ds(off, 16)]
        pos = btv >= THR
        posf = jnp.where(pos, 1.0, 0.0)
        ecx = ((m0 + m2) / 2 - cx) / (V0 * w)
        ecy = ((m1 + m3) / 2 - cy) / (V0 * h)
        ew = _ln(jnp.maximum((m2 - m0) / w, 1e-8)) / V1
        eh = _ln(jnp.maximum((m3 - m1) / h, 1e-8)) / V1
        s = _spl_f(0.0)
        for comp, enc in ((0, ecx), (1, ecy), (2, ew), (3, eh)):
            d = buf_loc[comp, pl.ds(off, 16)] - enc
            ad = jnp.abs(d)
            s = s + jnp.where(ad < 1.0, 0.5 * d * d, ad - 0.5)
        a_sl1 = a_sl1 + s * posf
        x0 = buf_cls[0, pl.ds(off, 16)]
        x1 = buf_cls[1, pl.ds(off, 16)]
        mx = jnp.maximum(x0, x1)
        z = jnp.exp(-jnp.abs(x0 - x1))
        lse = mx + _ln_m(1.0 + z)
        ce1 = lse - x1
        ce0 = lse - x0
        a_ce = a_ce + jnp.where(pos, ce1, 0.0)
        a_np = a_np + jnp.where(pos, 1, 0)
        lc[pl.ds(ci * CHUNK + off, 16)] = jnp.where(pos, _spl_f(-0.0), ce0)
        return a_sl1, a_ce, a_np

    def a2_body(ci, gbase, vi, st):
        bv8, bi8 = st
        off = vi * 16
        pidx = _spl_i(ci * CHUNK) + _spl_i(off) + it16
        cx = buf_pb[0, pl.ds(off, 16)]
        cy = buf_pb[1, pl.ds(off, 16)]
        w = buf_pb[2, pl.ds(off, 16)]
        h = buf_pb[3, pl.ds(off, 16)]
        px1 = cx - w / 2
        py1 = cy - h / 2
        px2 = cx + w / 2
        py2 = cy + h / 2
        parea = (px2 - px1) * (py2 - py1)
        bv2 = []
        bi2 = []
        for j in range(8):
            g = gbase + j
            g0 = gt_bc(0, g)
            g1 = gt_bc(1, g)
            g2 = gt_bc(2, g)
            g3 = gt_bc(3, g)
            ga = gt_bc(4, g)
            wi = jnp.maximum(jnp.minimum(g2, px2) - jnp.maximum(g0, px1), 0.0)
            hi = jnp.maximum(jnp.minimum(g3, py2) - jnp.maximum(g1, py1), 0.0)
            inter = wi * hi
            iou = inter / (ga + parea - inter)
            mg = iou > bv8[j]
            bv2.append(jnp.where(mg, iou, bv8[j]))
            bi2.append(jnp.where(mg, pidx, bi8[j]))
        return tuple(bv2), tuple(bi2)

    def chunk_body(ci, st):
        bv, bi, a_sl1, a_ce, a_np = st
        pltpu.sync_copy(pbt.at[:, pl.ds(ci * CHUNK, CHUNK)], buf_pb)
        pltpu.sync_copy(loct.at[wid, :, pl.ds(ci * CHUNK, CHUNK)], buf_loc)
        pltpu.sync_copy(clst.at[wid, :, pl.ds(ci * CHUNK, CHUNK)], buf_cls)
        bv, bi = lax.fori_loop(0, NVEC, functools.partial(a1_body, ci),
                               (bv, bi), unroll=2)
        a_sl1, a_ce, a_np = lax.fori_loop(
            0, NVEC, functools.partial(a1c_body, ci),
            (a_sl1, a_ce, a_np), unroll=2)
        return (bv, bi, a_sl1, a_ce, a_np)

    init = (tuple(_spl_f(-1.0) for _ in range(G)),
            tuple(_spl_i(0) for _ in range(G)),
            _spl_f(0.0), _spl_f(0.0), _spl_i(0))
    bv, bi, a_sl1, a_ce, a_np = lax.fori_loop(0, NCHUNK, chunk_body, init, unroll=False)

    bpi = _spl_i(0)
    pgs = []
    for g in range(G):
        gmax = jnp.max(bv[g])
        cand = jnp.where(bv[g] == gmax, bi[g], P)
        pg = jnp.min(cand)
        pgs.append(pg)
        bpi = jnp.where(it16 == g, pg, bpi)
    loser = it16 < 0
    for j in range(G):
        loser = loser | ((it16 < j) & (bpi == pgs[j]))
    winner = jnp.logical_not(loser)
    def fetch(hbm_flat, base):
        idx_ref[...] = bpi + _spl_i(base)
        pltpu.async_copy(hbm_flat.at[idx_ref], gvec, sem).wait()
        return gvec[...]

    pcx = fetch(pbf, 0)
    pcy = fetch(pbf, P)
    pw = fetch(pbf, 2 * P)
    ph = fetch(pbf, 3 * P)
    lp = [fetch(locf, (wid * 4 + c) * P) for c in range(4)]
    bx0 = fetch(clsf, wid * 2 * P)
    bx1 = fetch(clsf, (wid * 2 + 1) * P)
    px1 = pcx - pw / 2
    py1 = pcy - ph / 2
    px2 = pcx + pw / 2
    py2 = pcy + ph / 2
    parea = (px2 - px1) * (py2 - py1)
    obtv = _spl_f(-1.0)
    om0 = _spl_f(0.0)
    om1 = _spl_f(0.0)
    om2 = _spl_f(0.0)
    om3 = _spl_f(0.0)
    for g in range(G):
        g0 = gt_bc(0, g)
        g1 = gt_bc(1, g)
        g2 = gt_bc(2, g)
        g3 = gt_bc(3, g)
        ga = gt_bc(4, g)
        wi = jnp.maximum(jnp.minimum(g2, px2) - jnp.maximum(g0, px1), 0.0)
        hi = jnp.maximum(jnp.minimum(g3, py2) - jnp.maximum(g1, py1), 0.0)
        inter = wi * hi
        iou = inter / (ga + parea - inter)
        m = iou > obtv
        obtv = jnp.where(m, iou, obtv)
        om0 = jnp.where(m, g0, om0)
        om1 = jnp.where(m, g1, om1)
        om2 = jnp.where(m, g2, om2)
        om3 = jnp.where(m, g3, om3)
    old_pos = obtv >= THR

    def enc_sl1(m0, m1, m2, m3):
        ecx = ((m0 + m2) / 2 - pcx) / (V0 * pw)
        ecy = ((m1 + m3) / 2 - pcy) / (V0 * ph)
        ew = _ln(jnp.maximum((m2 - m0) / pw, 1e-8)) / V1
        eh = _ln(jnp.maximum((m3 - m1) / ph, 1e-8)) / V1
        s = _spl_f(0.0)
        for comp, enc in ((0, ecx), (1, ecy), (2, ew), (3, eh)):
            d = lp[comp] - enc
            ad = jnp.abs(d)
            s = s + jnp.where(ad < 1.0, 0.5 * d * d, ad - 0.5)
        return s

    sl1_new = enc_sl1(c0r, c1r, c2r, c3r)
    sl1_old = enc_sl1(om0, om1, om2, om3)
    z = jnp.exp(-jnp.abs(bx0 - bx1))
    lse = jnp.maximum(bx0, bx1) + _ln_m(1.0 + z)
    ce1 = lse - bx1
    newpos = winner & jnp.logical_not(old_pos)
    a_sl1 = a_sl1 + jnp.where(winner, sl1_new - jnp.where(old_pos, sl1_old, 0.0), 0.0)
    a_ce = a_ce + jnp.where(newpos, ce1, 0.0)
    a_np = a_np + jnp.where(newpos, 1, 0)
    plsc.store_scatter(lc, [bpi], _spl_f(-0.0), mask=winner)

    r_sl1 = jnp.sum(a_sl1)
    r_ce = jnp.sum(a_ce)
    r_np = jnp.sum(a_np)

    k = jnp.minimum(NEGPOS * r_np, P - 1)

    def zero_body(j, _):
        hist_cnt[pl.ds(j * 16, 16)] = _spl_i(0)
        hist_sum[pl.ds(j * 16, 16)] = _spl_f(0.0)
        return 0

    lax.fori_loop(0, NBIN // 16, zero_body, 0, unroll=False)

    def hist_body(j, _):
        v = lc[pl.ds(j * 16, 16)]
        mb = lax.bitcast_convert_type(v, i32) & 0x7FFFFFFF
        bn = mb >> 19
        plsc.addupdate_scatter(hist_cnt, [bn], _spl_i(1))
        plsc.addupdate_scatter(hist_sum, [bn], jnp.abs(v))
        return 0

    lax.fori_loop(0, P // 16, hist_body, 0, unroll=False)

    def scan_body(j, st):
        done, run_c, run_s, beta, cnt_hi, sum_hi = st
        jj = NBIN // 16 - 1 - j
        cv = hist_cnt[pl.ds(jj * 16, 16)]
        sv = hist_sum[pl.ds(jj * 16, 16)]
        s = jnp.sum(cv)
        found = jnp.logical_and(jnp.logical_not(done), run_c + s >= k)
        incl = plsc.cumsum(cv)
        cum_top = run_c + s - incl + cv
        maskl = cum_top >= k
        nb = _popcount(maskl)
        beta_lane = nb - 1
        beta_c = jj * 16 + beta_lane
        cnt_hi_c = run_c + jnp.sum(jnp.where(it16 > beta_lane, cv, 0))
        sum_hi_c = run_s + jnp.sum(jnp.where(it16 > beta_lane, sv, 0.0))
        beta = jnp.where(found, beta_c, beta)
        cnt_hi = jnp.where(found, cnt_hi_c, cnt_hi)
        sum_hi = jnp.where(found, sum_hi_c, sum_hi)
        done2 = jnp.logical_or(done, found)
        run_c = jnp.where(done2, run_c, run_c + s)
        run_s = jnp.where(done2, run_s, run_s + jnp.sum(sv))
        return done2, run_c, run_s, beta, cnt_hi, sum_hi

    _, _, _, beta, cnt_hi, sum_hi = lax.fori_loop(
        0, NBIN // 16, scan_body,
        (jnp.bool_(False), jnp.int32(0), jnp.float32(0.0),
         jnp.int32(0), jnp.int32(0), jnp.float32(0.0)), unroll=False)

    def subzero_body(j, _):
        subset[pl.ds(j * 16, 16)] = _spl_i(0)
        return 0

    lax.fori_loop(0, (SUBSET + 16) // 16, subzero_body, 0, unroll=False)

    def compact_body(j, off):
        v = lc[pl.ds(j * 16, 16)]
        mb = lax.bitcast_convert_type(v, i32) & 0x7FFFFFFF
        m = jnp.logical_and(mb >> 19 == beta, off < SUBSET)
        plsc.store_compressed(subset.at[pl.ds(off, 16)], mb, mask=m)
        return off + _popcount(m)

    s_cnt = lax.fori_loop(0, P // 16, compact_body, jnp.int32(0), unroll=False)
    nvec = (s_cnt + 15) // 16
    r = k - cnt_hi

    def count_ge(u):
        def cbody(i, acc):
            sv = subset[pl.ds(i * 16, 16)]
            valid = (it16 + i * 16) < s_cnt
            mm = jnp.logical_and(sv >= u, valid)
            return acc + plsc.all_reduce_population_count(mm)
        return jnp.max(lax.fori_loop(0, nvec, cbody, _spl_i(0)))

    lo = beta << 19
    hi = lo | 0x7FFFF
    for _ in range(19):
        mid = lo + ((hi - lo + 1) >> 1)
        ge = count_ge(mid) >= r
        lo = jnp.where(ge, mid, lo)
        hi = jnp.where(ge, hi, mid - 1)
    tbits = lo
    tval = lax.bitcast_convert_type(tbits, f32)

    def above_body(i, st):
        c, sacc = st
        sv = subset[pl.ds(i * 16, 16)]
        valid = (it16 + i * 16) < s_cnt
        mm = jnp.logical_and(sv > tbits, valid)
        c = c + plsc.all_reduce_population_count(mm)
        sacc = sacc + jnp.where(mm, lax.bitcast_convert_type(sv, f32), 0.0)
        return c, sacc

    c2v, s2v = lax.fori_loop(0, nvec, above_body, (_spl_i(0), _spl_f(0.0)))
    c2 = jnp.max(c2v)
    s2 = jnp.sum(s2v)
    cnt_gt = cnt_hi + c2
    sum_gt = sum_hi + s2
    topk = sum_gt + (k - cnt_gt).astype(f32) * tval
    numer = r_ce + topk

    def edge_fn(_):
        need = k - cnt_gt

        def ebody(j, st):
            run, extra = st
            v = lc[pl.ds(j * 16, 16)]
            bb = lax.bitcast_convert_type(v, i32)
            zm = (bb & 0x7FFFFFFF) == 0
            incl = plsc.cumsum(jnp.where(zm, 1, 0))
            sel = jnp.logical_and(zm, (run + incl) <= need)
            extra = extra + _popcount(jnp.logical_and(sel, bb == 0))
            return run + _popcount(zm), extra

        _, extra = lax.fori_loop(0, P // 16, ebody, (jnp.int32(0), jnp.int32(0)))
        return r_np + cnt_gt + extra

    mask_cnt = lax.cond(tbits == 0, edge_fn, lambda _: r_np + k, 0)

    o = jnp.where(it16 == 0, r_sl1,
                  jnp.where(it16 == 1, r_np.astype(f32),
                            jnp.where(it16 == 2, numer,
                                      jnp.where(it16 == 3, mask_cnt.astype(f32), 0.0))))
    outv[...] = o
    pltpu.sync_copy(outv, out.at[wid])


@jax.jit
def kernel(loc_preds, cls_preds, priorbox, targets):
    pbt = priorbox.T
    loct = jnp.transpose(loc_preds, (0, 2, 1))
    clst = jnp.transpose(cls_preds, (0, 2, 1))
    pbf = pbt.reshape(4 * P)
    locf = loct.reshape(B * 4 * P)
    clsf = clst.reshape(B * 2 * P)
    tgf = targets.reshape(B, G * 5)
    mesh = plsc.VectorSubcoreMesh(core_axis_name="c", subcore_axis_name="s",
                                  num_cores=NC, num_subcores=NS)
    out = pl.kernel(
        _sc_body,
        out_type=jax.ShapeDtypeStruct((B, 16), f32),
        mesh=mesh,
        compiler_params=pltpu.CompilerParams(needs_layout_passes=False),
        scratch_types=[
            pltpu.VMEM((4, CHUNK), f32),
            pltpu.VMEM((4, CHUNK), f32),
            pltpu.VMEM((2, CHUNK), f32),
            pltpu.VMEM((P,), f32),
            pltpu.VMEM((G * 5,), f32),
            pltpu.VMEM((5 * G * 16,), f32),
            pltpu.VMEM((NBIN,), i32),
            pltpu.VMEM((NBIN,), f32),
            pltpu.VMEM((SUBSET + 16,), i32),
            pltpu.VMEM((16,), f32),
            pltpu.VMEM((16,), i32),
            pltpu.VMEM((16,), f32),
            pltpu.VMEM((5, CHUNK), f32),
            pltpu.SemaphoreType.DMA,
        ],
    )(pbt, loct, clst, tgf, pbf, locf, clsf)
    tot = jnp.sum(out, axis=0)
    tot_sl1, tot_pos, tot_num, tot_mask = tot[0], tot[1], tot[2], tot[3]
    loss_loc = tot_sl1 / jnp.maximum(tot_pos * 4.0, 1.0)
    loss_cls = tot_num / jnp.maximum(tot_mask, 1.0)
    loss = (loss_cls + loss_loc) / jnp.maximum(tot_pos, 1.0)
    return (loss, loss_loc, loss_cls)


# speedup vs baseline: 70.4824x; 1.0049x over previous
import functools

import jax
import jax.numpy as jnp
from jax import lax
from jax.experimental import pallas as pl
from jax.experimental.pallas import tpu as pltpu
from jax.experimental.pallas import tpu_sc as plsc

B = 32
P = 32768
G = 16
NC = 2
NS = 16
CHUNK = 4096
NCHUNK = P // CHUNK
NVEC = CHUNK // 16
THR = 0.35
V0, V1 = 0.1, 0.2
NEGPOS = 3
NBIN = 4096
SUBSET = 8192
LN2 = 0.6931471805599453

f32 = jnp.float32
i32 = jnp.int32


def _i16():
    return lax.iota(i32, 16)


def _spl_f(x):
    return jnp.full((16,), x, f32)


def _spl_i(x):
    return jnp.full((16,), x, i32)


def _ln_m(x):
    s = (x - 1.0) / (x + 1.0)
    s2 = s * s
    return s * (2.0 + s2 * (2.0 / 3.0 + s2 * (2.0 / 5.0 + s2 * (2.0 / 7.0 + s2 * (2.0 / 9.0)))))


def _ln(x):
    b = lax.bitcast_convert_type(x, i32)
    e = (b >> 23) - 127
    m = lax.bitcast_convert_type((b & 0x7FFFFF) | 0x3F800000, f32)
    return e.astype(f32) * LN2 + _ln_m(m)


def _popcount(mask):
    return jnp.max(plsc.all_reduce_population_count(mask))


def _sc_body(pbt, loct, clst, tgt, pbf, locf, clsf, out,
             buf_pb, buf_loc, buf_cls, lc, tgv, gtc, hist_cnt, hist_sum,
             subset, gvec, idx_ref, outv, mbuf, sem):
    wid = lax.axis_index("s") * NC + lax.axis_index("c")
    it16 = _i16()

    pltpu.sync_copy(tgt.at[wid], tgv)
    tw = [tgv[pl.ds(o * 16, 16)] for o in range(5)]

    def tsc(j):
        return tw[j // 16][j % 16]

    c0r = _spl_f(0.0)
    c1r = _spl_f(0.0)
    c2r = _spl_f(0.0)
    c3r = _spl_f(0.0)
    for g in range(G):
        t0 = tsc(5 * g)
        t1 = tsc(5 * g + 1)
        t2 = tsc(5 * g + 2)
        t3 = tsc(5 * g + 3)
        mg = it16 == g
        c0r = jnp.where(mg, t1, c0r)
        c1r = jnp.where(mg, t0, c1r)
        c2r = jnp.where(mg, t3, c2r)
        c3r = jnp.where(mg, t2, c3r)
        ag = (t3 - t1) * (t2 - t0)
        gtc[pl.ds((0 * G + g) * 16, 16)] = jnp.full((16,), t1, f32)
        gtc[pl.ds((1 * G + g) * 16, 16)] = jnp.full((16,), t0, f32)
        gtc[pl.ds((2 * G + g) * 16, 16)] = jnp.full((16,), t3, f32)
        gtc[pl.ds((3 * G + g) * 16, 16)] = jnp.full((16,), t2, f32)
        gtc[pl.ds((4 * G + g) * 16, 16)] = jnp.full((16,), ag, f32)
    def gt_bc(comp, g):
        return gtc[pl.ds((comp * G + g) * 16, 16)]

    def a1_body(ci, vi, st):
        bv8, bi8 = st
        off = vi * 16
        pidx = _spl_i(ci * CHUNK) + _spl_i(off) + it16
        cx = buf_pb[0, pl.ds(off, 16)]
        cy = buf_pb[1, pl.ds(off, 16)]
        w = buf_pb[2, pl.ds(off, 16)]
        h = buf_pb[3, pl.ds(off, 16)]
        px1 = cx - w / 2
        py1 = cy - h / 2
        px2 = cx + w / 2
        py2 = cy + h / 2
        parea = (px2 - px1) * (py2 - py1)
        btv = _spl_f(-1.0)
        m0 = _spl_f(0.0)
        m1 = _spl_f(0.0)
        m2 = _spl_f(0.0)
        m3 = _spl_f(0.0)
        bv2 = []
        bi2 = []
        for g in range(G):
            g0 = gt_bc(0, g)
            g1 = gt_bc(1, g)
            g2 = gt_bc(2, g)
            g3 = gt_bc(3, g)
            ga = gt_bc(4, g)
            wi = jnp.maximum(jnp.minimum(g2, px2) - jnp.maximum(g0, px1), 0.0)
            hi = jnp.maximum(jnp.minimum(g3, py2) - jnp.maximum(g1, py1), 0.0)
            inter = wi * hi
            iou = inter / (ga + parea - inter)
            m = iou > btv
            btv = jnp.where(m, iou, btv)
            m0 = jnp.where(m, g0, m0)
            m1 = jnp.where(m, g1, m1)
            m2 = jnp.where(m, g2, m2)
            m3 = jnp.where(m, g3, m3)
            if True:
                mg = iou > bv8[g]
                bv2.append(jnp.where(mg, iou, bv8[g]))
                bi2.append(jnp.where(mg, pidx, bi8[g]))
        mbuf[0, pl.ds(off, 16)] = btv
        mbuf[1, pl.ds(off, 16)] = m0
        mbuf[2, pl.ds(off, 16)] = m1
        mbuf[3, pl.ds(off, 16)] = m2
        mbuf[4, pl.ds(off, 16)] = m3
        return tuple(bv2), tuple(bi2)

    def a1c_body(ci, vi, st):
        a_sl1, a_ce, a_np = st
        off = vi * 16
        cx = buf_pb[0, pl.ds(off, 16)]
        cy = buf_pb[1, pl.ds(off, 16)]
        w = buf_pb[2, pl.ds(off, 16)]
        h = buf_pb[3, pl.ds(off, 16)]
        btv = mbuf[0, pl.ds(off, 16)]
        m0 = mbuf[1, pl.ds(off, 16)]
        m1 = mbuf[2, pl.ds(off, 16)]
        m2 = mbuf[3, pl.ds(off, 16)]
        m3 = mbuf[4, pl.ds(off, 16)]
        pos = btv >= THR
        posf = jnp.where(pos, 1.0, 0.0)
        ecx = ((m0 + m2) / 2 - cx) / (V0 * w)
        ecy = ((m1 + m3) / 2 - cy) / (V0 * h)
        ew = _ln(jnp.maximum((m2 - m0) / w, 1e-8)) / V1
        eh = _ln(jnp.maximum((m3 - m1) / h, 1e-8)) / V1
        s = _spl_f(0.0)
        for comp, enc in ((0, ecx), (1, ecy), (2, ew), (3, eh)):
            d = buf_loc[comp, pl.ds(off, 16)] - enc
            ad = jnp.abs(d)
            s = s + jnp.where(ad < 1.0, 0.5 * d * d, ad - 0.5)
        a_sl1 = a_sl1 + s * posf
        x0 = buf_cls[0, pl.ds(off, 16)]
        x1 = buf_cls[1, pl.ds(off, 16)]
        mx = jnp.maximum(x0, x1)
        z = jnp.exp(-jnp.abs(x0 - x1))
        lse = mx + _ln_m(1.0 + z)
        ce1 = lse - x1
        ce0 = lse - x0
        a_ce = a_ce + jnp.where(pos, ce1, 0.0)
        a_np = a_np + jnp.where(pos, 1, 0)
        lc[pl.ds(ci * CHUNK + off, 16)] = jnp.where(pos, _spl_f(-0.0), ce0)
        return a_sl1, a_ce, a_np

    def a2_body(ci, gbase, vi, st):
        bv8, bi8 = st
        off = vi * 16
        pidx = _spl_i(ci * CHUNK) + _spl_i(off) + it16
        cx = buf_pb[0, pl.ds(off, 16)]
        cy = buf_pb[1, pl.ds(off, 16)]
        w = buf_pb[2, pl.ds(off, 16)]
        h = buf_pb[3, pl.ds(off, 16)]
        px1 = cx - w / 2
        py1 = cy - h / 2
        px2 = cx + w / 2
        py2 = cy + h / 2
        parea = (px2 - px1) * (py2 - py1)
        bv2 = []
        bi2 = []
        for j in range(8):
            g = gbase + j
            g0 = gt_bc(0, g)
            g1 = gt_bc(1, g)
            g2 = gt_bc(2, g)
            g3 = gt_bc(3, g)
            ga = gt_bc(4, g)
            wi = jnp.maximum(jnp.minimum(g2, px2) - jnp.maximum(g0, px1), 0.0)
            hi = jnp.maximum(jnp.minimum(g3, py2) - jnp.maximum(g1, py1), 0.0)
            inter = wi * hi
            iou = inter / (ga + parea - inter)
            mg = iou > bv8[j]
            bv2.append(jnp.where(mg, iou, bv8[j]))
            bi2.append(jnp.where(mg, pidx, bi8[j]))
        return tuple(bv2), tuple(bi2)

    def chunk_body(ci, st):
        bv, bi, a_sl1, a_ce, a_np = st
        pltpu.sync_copy(pbt.at[:, pl.ds(ci * CHUNK, CHUNK)], buf_pb)
        pltpu.sync_copy(loct.at[wid, :, pl.ds(ci * CHUNK, CHUNK)], buf_loc)
        pltpu.sync_copy(clst.at[wid, :, pl.ds(ci * CHUNK, CHUNK)], buf_cls)
        bv, bi = lax.fori_loop(0, NVEC, functools.partial(a1_body, ci),
                               (bv, bi), unroll=2)
        a_sl1, a_ce, a_np = lax.fori_loop(
            0, NVEC, functools.partial(a1c_body, ci),
            (a_sl1, a_ce, a_np), unroll=3)
        return (bv, bi, a_sl1, a_ce, a_np)

    init = (tuple(_spl_f(-1.0) for _ in range(G)),
            tuple(_spl_i(0) for _ in range(G)),
            _spl_f(0.0), _spl_f(0.0), _spl_i(0))
    bv, bi, a_sl1, a_ce, a_np = lax.fori_loop(0, NCHUNK, chunk_body, init, unroll=False)

    bpi = _spl_i(0)
    pgs = []
    for g in range(G):
        gmax = jnp.max(bv[g])
        cand = jnp.where(bv[g] == gmax, bi[g], P)
        pg = jnp.min(cand)
        pgs.append(pg)
        bpi = jnp.where(it16 == g, pg, bpi)
    loser = it16 < 0
    for j in range(G):
        loser = loser | ((it16 < j) & (bpi == pgs[j]))
    winner = jnp.logical_not(loser)
    def fetch(hbm_flat, base):
        idx_ref[...] = bpi + _spl_i(base)
        pltpu.async_copy(hbm_flat.at[idx_ref], gvec, sem).wait()
        return gvec[...]

    pcx = fetch(pbf, 0)
    pcy = fetch(pbf, P)
    pw = fetch(pbf, 2 * P)
    ph = fetch(pbf, 3 * P)
    lp = [fetch(locf, (wid * 4 + c) * P) for c in range(4)]
    bx0 = fetch(clsf, wid * 2 * P)
    bx1 = fetch(clsf, (wid * 2 + 1) * P)
    px1 = pcx - pw / 2
    py1 = pcy - ph / 2
    px2 = pcx + pw / 2
    py2 = pcy + ph / 2
    parea = (px2 - px1) * (py2 - py1)
    obtv = _spl_f(-1.0)
    om0 = _spl_f(0.0)
    om1 = _spl_f(0.0)
    om2 = _spl_f(0.0)
    om3 = _spl_f(0.0)
    for g in range(G):
        g0 = gt_bc(0, g)
        g1 = gt_bc(1, g)
        g2 = gt_bc(2, g)
        g3 = gt_bc(3, g)
        ga = gt_bc(4, g)
        wi = jnp.maximum(jnp.minimum(g2, px2) - jnp.maximum(g0, px1), 0.0)
        hi = jnp.maximum(jnp.minimum(g3, py2) - jnp.maximum(g1, py1), 0.0)
        inter = wi * hi
        iou = inter / (ga + parea - inter)
        m = iou > obtv
        obtv = jnp.where(m, iou, obtv)
        om0 = jnp.where(m, g0, om0)
        om1 = jnp.where(m, g1, om1)
        om2 = jnp.where(m, g2, om2)
        om3 = jnp.where(m, g3, om3)
    old_pos = obtv >= THR

    def enc_sl1(m0, m1, m2, m3):
        ecx = ((m0 + m2) / 2 - pcx) / (V0 * pw)
        ecy = ((m1 + m3) / 2 - pcy) / (V0 * ph)
        ew = _ln(jnp.maximum((m2 - m0) / pw, 1e-8)) / V1
        eh = _ln(jnp.maximum((m3 - m1) / ph, 1e-8)) / V1
        s = _spl_f(0.0)
        for comp, enc in ((0, ecx), (1, ecy), (2, ew), (3, eh)):
            d = lp[comp] - enc
            ad = jnp.abs(d)
            s = s + jnp.where(ad < 1.0, 0.5 * d * d, ad - 0.5)
        return s

    sl1_new = enc_sl1(c0r, c1r, c2r, c3r)
    sl1_old = enc_sl1(om0, om1, om2, om3)
    z = jnp.exp(-jnp.abs(bx0 - bx1))
    lse = jnp.maximum(bx0, bx1) + _ln_m(1.0 + z)
    ce1 = lse - bx1
    newpos = winner & jnp.logical_not(old_pos)
    a_sl1 = a_sl1 + jnp.where(winner, sl1_new - jnp.where(old_pos, sl1_old, 0.0), 0.0)
    a_ce = a_ce + jnp.where(newpos, ce1, 0.0)
    a_np = a_np + jnp.where(newpos, 1, 0)
    plsc.store_scatter(lc, [bpi], _spl_f(-0.0), mask=winner)

    r_sl1 = jnp.sum(a_sl1)
    r_ce = jnp.sum(a_ce)
    r_np = jnp.sum(a_np)

    k = jnp.minimum(NEGPOS * r_np, P - 1)

    def zero_body(j, _):
        hist_cnt[pl.ds(j * 16, 16)] = _spl_i(0)
        hist_sum[pl.ds(j * 16, 16)] = _spl_f(0.0)
        return 0

    lax.fori_loop(0, NBIN // 16, zero_body, 0, unroll=False)

    def hist_body(j, _):
        v = lc[pl.ds(j * 16, 16)]
        mb = lax.bitcast_convert_type(v, i32) & 0x7FFFFFFF
        bn = mb >> 19
        plsc.addupdate_scatter(hist_cnt, [bn], _spl_i(1))
        plsc.addupdate_scatter(hist_sum, [bn], jnp.abs(v))
        return 0

    lax.fori_loop(0, P // 16, hist_body, 0, unroll=False)

    def scan_body(j, st):
        done, run_c, run_s, beta, cnt_hi, sum_hi = st
        jj = NBIN // 16 - 1 - j
        cv = hist_cnt[pl.ds(jj * 16, 16)]
        sv = hist_sum[pl.ds(jj * 16, 16)]
        s = jnp.sum(cv)
        found = jnp.logical_and(jnp.logical_not(done), run_c + s >= k)
        incl = plsc.cumsum(cv)
        cum_top = run_c + s - incl + cv
        maskl = cum_top >= k
        nb = _popcount(maskl)
        beta_lane = nb - 1
        beta_c = jj * 16 + beta_lane
        cnt_hi_c = run_c + jnp.sum(jnp.where(it16 > beta_lane, cv, 0))
        sum_hi_c = run_s + jnp.sum(jnp.where(it16 > beta_lane, sv, 0.0))
        beta = jnp.where(found, beta_c, beta)
        cnt_hi = jnp.where(found, cnt_hi_c, cnt_hi)
        sum_hi = jnp.where(found, sum_hi_c, sum_hi)
        done2 = jnp.logical_or(done, found)
        run_c = jnp.where(done2, run_c, run_c + s)
        run_s = jnp.where(done2, run_s, run_s + jnp.sum(sv))
        return done2, run_c, run_s, beta, cnt_hi, sum_hi

    _, _, _, beta, cnt_hi, sum_hi = lax.fori_loop(
        0, NBIN // 16, scan_body,
        (jnp.bool_(False), jnp.int32(0), jnp.float32(0.0),
         jnp.int32(0), jnp.int32(0), jnp.float32(0.0)), unroll=False)

    def subzero_body(j, _):
        subset[pl.ds(j * 16, 16)] = _spl_i(0)
        return 0

    lax.fori_loop(0, (SUBSET + 16) // 16, subzero_body, 0, unroll=False)

    def compact_body(j, off):
        v = lc[pl.ds(j * 16, 16)]
        mb = lax.bitcast_convert_type(v, i32) & 0x7FFFFFFF
        m = jnp.logical_and(mb >> 19 == beta, off < SUBSET)
        plsc.store_compressed(subset.at[pl.ds(off, 16)], mb, mask=m)
        return off + _popcount(m)

    s_cnt = lax.fori_loop(0, P // 16, compact_body, jnp.int32(0), unroll=False)
    nvec = (s_cnt + 15) // 16
    r = k - cnt_hi

    def count_ge(u):
        def cbody(i, acc):
            sv = subset[pl.ds(i * 16, 16)]
            valid = (it16 + i * 16) < s_cnt
            mm = jnp.logical_and(sv >= u, valid)
            return acc + plsc.all_reduce_population_count(mm)
        return jnp.max(lax.fori_loop(0, nvec, cbody, _spl_i(0)))

    lo = beta << 19
    hi = lo | 0x7FFFF
    for _ in range(19):
        mid = lo + ((hi - lo + 1) >> 1)
        ge = count_ge(mid) >= r
        lo = jnp.where(ge, mid, lo)
        hi = jnp.where(ge, hi, mid - 1)
    tbits = lo
    tval = lax.bitcast_convert_type(tbits, f32)

    def above_body(i, st):
        c, sacc = st
        sv = subset[pl.ds(i * 16, 16)]
        valid = (it16 + i * 16) < s_cnt
        mm = jnp.logical_and(sv > tbits, valid)
        c = c + plsc.all_reduce_population_count(mm)
        sacc = sacc + jnp.where(mm, lax.bitcast_convert_type(sv, f32), 0.0)
        return c, sacc

    c2v, s2v = lax.fori_loop(0, nvec, above_body, (_spl_i(0), _spl_f(0.0)))
    c2 = jnp.max(c2v)
    s2 = jnp.sum(s2v)
    cnt_gt = cnt_hi + c2
    sum_gt = sum_hi + s2
    topk = sum_gt + (k - cnt_gt).astype(f32) * tval
    numer = r_ce + topk

    def edge_fn(_):
        need = k - cnt_gt

        def ebody(j, st):
            run, extra = st
            v = lc[pl.ds(j * 16, 16)]
            bb = lax.bitcast_convert_type(v, i32)
            zm = (bb & 0x7FFFFFFF) == 0
            incl = plsc.cumsum(jnp.where(zm, 1, 0))
            sel = jnp.logical_and(zm, (run + incl) <= need)
            extra = extra + _popcount(jnp.logical_and(sel, bb == 0))
            return run + _popcount(zm), extra

        _, extra = lax.fori_loop(0, P // 16, ebody, (jnp.int32(0), jnp.int32(0)))
        return r_np + cnt_gt + extra

    mask_cnt = lax.cond(tbits == 0, edge_fn, lambda _: r_np + k, 0)

    o = jnp.where(it16 == 0, r_sl1,
                  jnp.where(it16 == 1, r_np.astype(f32),
                            jnp.where(it16 == 2, numer,
                                      jnp.where(it16 == 3, mask_cnt.astype(f32), 0.0))))
    outv[...] = o
    pltpu.sync_copy(outv, out.at[wid])


@jax.jit
def kernel(loc_preds, cls_preds, priorbox, targets):
    pbt = priorbox.T
    loct = jnp.transpose(loc_preds, (0, 2, 1))
    clst = jnp.transpose(cls_preds, (0, 2, 1))
    pbf = pbt.reshape(4 * P)
    locf = loct.reshape(B * 4 * P)
    clsf = clst.reshape(B * 2 * P)
    tgf = targets.reshape(B, G * 5)
    mesh = plsc.VectorSubcoreMesh(core_axis_name="c", subcore_axis_name="s",
                                  num_cores=NC, num_subcores=NS)
    out = pl.kernel(
        _sc_body,
        out_type=jax.ShapeDtypeStruct((B, 16), f32),
        mesh=mesh,
        compiler_params=pltpu.CompilerParams(needs_layout_passes=False),
        scratch_types=[
            pltpu.VMEM((4, CHUNK), f32),
            pltpu.VMEM((4, CHUNK), f32),
            pltpu.VMEM((2, CHUNK), f32),
            pltpu.VMEM((P,), f32),
            pltpu.VMEM((G * 5,), f32),
            pltpu.VMEM((5 * G * 16,), f32),
            pltpu.VMEM((NBIN,), i32),
            pltpu.VMEM((NBIN,), f32),
            pltpu.VMEM((SUBSET + 16,), i32),
            pltpu.VMEM((16,), f32),
            pltpu.VMEM((16,), i32),
            pltpu.VMEM((16,), f32),
            pltpu.VMEM((5, CHUNK), f32),
            pltpu.SemaphoreType.DMA,
        ],
    )(pbt, loct, clst, tgf, pbf, locf, clsf)
    tot = jnp.sum(out, axis=0)
    tot_sl1, tot_pos, tot_num, tot_mask = tot[0], tot[1], tot[2], tot[3]
    loss_loc = tot_sl1 / jnp.maximum(tot_pos * 4.0, 1.0)
    loss_cls = tot_num / jnp.maximum(tot_mask, 1.0)
    loss = (loss_cls + loss_loc) / jnp.maximum(tot_pos, 1.0)
    return (loss, loss_loc, loss_cls)
